# Initial kernel scaffold; baseline (speedup 1.0000x reference)
import jax
import jax.numpy as jnp
from jax.experimental import pallas as pl


def kernel(dNodeAttr, dEdgeAttr, node_tables, edge_tables):
    raise NotImplementedError("write your pallas kernel here")


# speedup vs baseline: 1.1241x; 1.1241x over previous
import jax
import jax.numpy as jnp
from jax import lax
from jax.experimental import pallas as pl
from jax.experimental.pallas import tpu as pltpu
from jax.experimental.pallas import tpu_sc as plsc

_ATOM_DIMS = (119, 5, 12, 12, 10, 6, 6, 2, 2)
_BOND_DIMS = (5, 6, 2)
_D = 128
_N_NODES = 10000
_N_EDGES = 320000
_NF = len(_ATOM_DIMS)
_EF = len(_BOND_DIMS)

_NODE_OFF = tuple(int(sum(_ATOM_DIMS[:i])) for i in range(_NF))
_NCAT_ROWS = int(sum(_ATOM_DIMS))
_ECAT_ROWS = int(sum(_BOND_DIMS))
_ECOMB_ROWS = _BOND_DIMS[0] * _BOND_DIMS[1] * _BOND_DIMS[2]

_NC = 2
_NS = 16
_NW = _NC * _NS

_EPW = _N_EDGES // _NW
_ECHUNK = 400
_ENCHUNKS = _EPW // _ECHUNK

_NPW = 400
_N_NODE_WORKERS = _N_NODES // _NPW

_LANES = 16


def _splat(v, n=_LANES):
    return jnp.full((n,), v, jnp.int32)


def _sc_body(nidx_hbm, eidx_hbm, ncat_hbm, ecat_hbm, vnode_hbm, vedge_hbm,
             ncat_v, ecat_v, ecomb_v, eidx_v, nidx_v, out_v):
    wid = lax.axis_index("s") * _NC + lax.axis_index("c")
    iota = lax.iota(jnp.int32, _LANES)

    pltpu.sync_copy(ncat_hbm, ncat_v)
    pltpu.sync_copy(ecat_hbm, ecat_v)

    def build_comb(t, carry):
        a = t // 12
        r = t % 12
        b = r // 2
        c = r % 2
        rowa = _splat(a * _D)
        rowb = _splat((b + _BOND_DIMS[0]) * _D)
        rowc = _splat((c + _BOND_DIMS[0] + _BOND_DIMS[1]) * _D)
        rowt = _splat(t * _D)
        for j in range(_D // _LANES):
            col = iota + (j * _LANES)
            v = (plsc.load_gather(ecat_v, [rowa + col])
                 + plsc.load_gather(ecat_v, [rowb + col])
                 + plsc.load_gather(ecat_v, [rowc + col]))
            plsc.store_scatter(ecomb_v, [rowt + col], v)
        return carry

    lax.fori_loop(0, _ECOMB_ROWS, build_comb, 0)

    def edge_chunk(ch, carry):
        base = wid * _EPW + ch * _ECHUNK
        pltpu.sync_copy(eidx_hbm.at[pl.ds(base * _EF, _ECHUNK * _EF)], eidx_v)

        def group(g, carry2):
            rows = iota + g * _LANES
            i0 = plsc.load_gather(eidx_v, [rows * _EF])
            i1 = plsc.load_gather(eidx_v, [rows * _EF + 1])
            i2 = plsc.load_gather(eidx_v, [rows * _EF + 2])
            cidb = (i0 * 12 + i1 * 2 + i2) * _D
            rowb = rows * _D
            for j in range(_D):
                v = plsc.load_gather(ecomb_v, [cidb + j])
                plsc.store_scatter(out_v, [rowb + j], v)
            return carry2

        lax.fori_loop(0, _ECHUNK // _LANES, group, 0)
        pltpu.sync_copy(out_v.at[pl.ds(0, _ECHUNK * _D)],
                        vedge_hbm.at[pl.ds(base * _D, _ECHUNK * _D)])
        return carry

    lax.fori_loop(0, _ENCHUNKS, edge_chunk, 0)

    @pl.when(wid < _N_NODE_WORKERS)
    def node_phase():
        base = wid * _NPW
        pltpu.sync_copy(nidx_hbm.at[pl.ds(base * _NF, _NPW * _NF)], nidx_v)

        def group(g, carry2):
            rows = iota + g * _LANES
            ivs = [
                (plsc.load_gather(nidx_v, [rows * _NF + i]) + _NODE_OFF[i]) * _D
                for i in range(_NF)
            ]
            rowb = rows * _D
            for j in range(_D):
                acc = plsc.load_gather(ncat_v, [ivs[0] + j])
                for i in range(1, _NF):
                    acc = acc + plsc.load_gather(ncat_v, [ivs[i] + j])
                plsc.store_scatter(out_v, [rowb + j], acc)
            return carry2

        lax.fori_loop(0, _NPW // _LANES, group, 0)
        pltpu.sync_copy(out_v.at[pl.ds(0, _NPW * _D)],
                        vnode_hbm.at[pl.ds(base * _D, _NPW * _D)])


@jax.jit
def _sc_call(nidx_flat, eidx_flat, ncat_flat, ecat_flat):
    mesh = plsc.VectorSubcoreMesh(core_axis_name="c", subcore_axis_name="s")
    f = pl.kernel(
        _sc_body,
        out_type=(
            jax.ShapeDtypeStruct((_N_NODES * _D,), jnp.float32),
            jax.ShapeDtypeStruct((_N_EDGES * _D,), jnp.float32),
        ),
        mesh=mesh,
        compiler_params=pltpu.CompilerParams(needs_layout_passes=False),
        scratch_types=[
            pltpu.VMEM((_NCAT_ROWS * _D,), jnp.float32),
            pltpu.VMEM((_ECAT_ROWS * _D,), jnp.float32),
            pltpu.VMEM((_ECOMB_ROWS * _D,), jnp.float32),
            pltpu.VMEM((_ECHUNK * _EF,), jnp.int32),
            pltpu.VMEM((_NPW * _NF,), jnp.int32),
            pltpu.VMEM((max(_ECHUNK, _NPW) * _D,), jnp.float32),
        ],
    )
    return f(nidx_flat, eidx_flat, ncat_flat, ecat_flat)


def kernel(dNodeAttr, dEdgeAttr, node_tables, edge_tables):
    ncat = jnp.concatenate(node_tables, axis=0).reshape(-1)
    ecat = jnp.concatenate(edge_tables, axis=0).reshape(-1)
    vnode, vedge = _sc_call(dNodeAttr.reshape(-1), dEdgeAttr.reshape(-1),
                            ncat, ecat)
    return (vnode.reshape(_N_NODES, _D), vedge.reshape(_N_EDGES, _D))


# speedup vs baseline: 3.4667x; 3.0840x over previous
import jax
import jax.numpy as jnp
from jax import lax
from jax.experimental import pallas as pl
from jax.experimental.pallas import tpu as pltpu
from jax.experimental.pallas import tpu_sc as plsc

_ATOM_DIMS = (119, 5, 12, 12, 10, 6, 6, 2, 2)
_BOND_DIMS = (5, 6, 2)
_D = 128
_N_NODES = 10000
_N_EDGES = 320000
_NF = len(_ATOM_DIMS)
_EF = len(_BOND_DIMS)

_NODE_OFF = tuple(int(sum(_ATOM_DIMS[:i])) for i in range(_NF))
_NCAT_ROWS = int(sum(_ATOM_DIMS))
_ECAT_ROWS = int(sum(_BOND_DIMS))
_ECOMB_ROWS = _BOND_DIMS[0] * _BOND_DIMS[1] * _BOND_DIMS[2]

_NC = 2
_NS = 16
_NW = _NC * _NS

_EPW = _N_EDGES // _NW
_ECHUNK = 400
_ENCHUNKS = _EPW // _ECHUNK

_NPW = 400
_N_NODE_WORKERS = _N_NODES // _NPW
_NCHUNK = 400
_NNCHUNKS = _NPW // _NCHUNK

_LANES = 16


def _splat(v, n=_LANES):
    return jnp.full((n,), v, jnp.int32)


def _sc_body(nidx_hbm, eidx_hbm, ncat_hbm, ecat_hbm, vnode_hbm, vedge_hbm,
             ncat_v, ecat_v, ecomb_v, eidx_v, nidx_v, out_v):
    wid = lax.axis_index("s") * _NC + lax.axis_index("c")
    iota = lax.iota(jnp.int32, _LANES)

    def diag(k):
        return jnp.bitwise_and(iota + k, _LANES - 1)

    pltpu.sync_copy(ncat_hbm, ncat_v)
    pltpu.sync_copy(ecat_hbm, ecat_v)

    def build_comb(t, carry):
        a = t // 12
        r = t % 12
        b = r // 2
        c = r % 2
        rowa = _splat(a * _D)
        rowb = _splat((b + _BOND_DIMS[0]) * _D)
        rowc = _splat((c + _BOND_DIMS[0] + _BOND_DIMS[1]) * _D)
        rowt = _splat(t * _D)
        for j in range(_D // _LANES):
            col = iota + (j * _LANES)
            v = (plsc.load_gather(ecat_v, [rowa + col])
                 + plsc.load_gather(ecat_v, [rowb + col])
                 + plsc.load_gather(ecat_v, [rowc + col]))
            plsc.store_scatter(ecomb_v, [rowt + col], v)
        return carry

    lax.fori_loop(0, _ECOMB_ROWS, build_comb, 0)

    def edge_chunk(ch, carry):
        base = wid * _EPW + ch * _ECHUNK
        pltpu.sync_copy(eidx_hbm.at[pl.ds(base * _EF, _ECHUNK * _EF)], eidx_v)

        def group(g, carry2):
            rows = iota + g * _LANES
            i0 = plsc.load_gather(eidx_v, [rows * _EF])
            i1 = plsc.load_gather(eidx_v, [rows * _EF + 1])
            i2 = plsc.load_gather(eidx_v, [rows * _EF + 2])
            cidb = (i0 * 12 + i1 * 2 + i2) * _D
            rowb = rows * _D
            def col_block(j, carry3):
                src = cidb + j * _LANES
                dst = rowb + j * _LANES
                for k in range(_LANES):
                    d = diag(k)
                    v = plsc.load_gather(ecomb_v, [src + d])
                    plsc.store_scatter(out_v, [dst + d], v)
                return carry3

            lax.fori_loop(0, _D // _LANES, col_block, 0)
            return carry2

        lax.fori_loop(0, _ECHUNK // _LANES, group, 0)
        pltpu.sync_copy(out_v.at[pl.ds(0, _ECHUNK * _D)],
                        vedge_hbm.at[pl.ds(base * _D, _ECHUNK * _D)])
        return carry

    lax.fori_loop(0, _ENCHUNKS, edge_chunk, 0)

    def node_chunk(ch, carry):
        base = jnp.int32(wid * _NPW) + ch * _NCHUNK
        pltpu.sync_copy(nidx_hbm.at[pl.ds(base * _NF, _NCHUNK * _NF)], nidx_v)

        def group(g, carry2):
            rows = iota + g * _LANES
            ivs = [
                (plsc.load_gather(nidx_v, [rows * _NF + i]) + _NODE_OFF[i]) * _D
                for i in range(_NF)
            ]
            rowb = rows * _D

            def col_block(j, carry3):
                dst = rowb + j * _LANES
                jl = j * _LANES
                for k in range(_LANES):
                    d = diag(k)
                    off = jl + d
                    acc = plsc.load_gather(ncat_v, [ivs[0] + off])
                    for i in range(1, _NF):
                        acc = acc + plsc.load_gather(ncat_v, [ivs[i] + off])
                    plsc.store_scatter(out_v, [dst + d], acc)
                return carry3

            lax.fori_loop(0, _D // _LANES, col_block, 0)
            return carry2

        lax.fori_loop(0, _NCHUNK // _LANES, group, 0)
        pltpu.sync_copy(out_v.at[pl.ds(0, _NCHUNK * _D)],
                        vnode_hbm.at[pl.ds(base * _D, _NCHUNK * _D)])
        return carry

    @pl.when(wid < _N_NODE_WORKERS)
    def node_phase():
        lax.fori_loop(0, _NNCHUNKS, node_chunk, 0)


@jax.jit
def _sc_call(nidx_flat, eidx_flat, ncat_flat, ecat_flat):
    mesh = plsc.VectorSubcoreMesh(core_axis_name="c", subcore_axis_name="s")
    f = pl.kernel(
        _sc_body,
        out_type=(
            jax.ShapeDtypeStruct((_N_NODES * _D,), jnp.float32),
            jax.ShapeDtypeStruct((_N_EDGES * _D,), jnp.float32),
        ),
        mesh=mesh,
        compiler_params=pltpu.CompilerParams(needs_layout_passes=False),
        scratch_types=[
            pltpu.VMEM((_NCAT_ROWS * _D,), jnp.float32),
            pltpu.VMEM((_ECAT_ROWS * _D,), jnp.float32),
            pltpu.VMEM((_ECOMB_ROWS * _D,), jnp.float32),
            pltpu.VMEM((_ECHUNK * _EF,), jnp.int32),
            pltpu.VMEM((_NCHUNK * _NF,), jnp.int32),
            pltpu.VMEM((max(_ECHUNK, _NCHUNK) * _D,), jnp.float32),
        ],
    )
    return f(nidx_flat, eidx_flat, ncat_flat, ecat_flat)


def kernel(dNodeAttr, dEdgeAttr, node_tables, edge_tables):
    ncat = jnp.concatenate(node_tables, axis=0).reshape(-1)
    ecat = jnp.concatenate(edge_tables, axis=0).reshape(-1)
    vnode, vedge = _sc_call(dNodeAttr.reshape(-1), dEdgeAttr.reshape(-1),
                            ncat, ecat)
    return (vnode.reshape(_N_NODES, _D), vedge.reshape(_N_EDGES, _D))


# speedup vs baseline: 3.8373x; 1.1069x over previous
import jax
import jax.numpy as jnp
from jax import lax
from jax.experimental import pallas as pl
from jax.experimental.pallas import tpu as pltpu
from jax.experimental.pallas import tpu_sc as plsc

_ATOM_DIMS = (119, 5, 12, 12, 10, 6, 6, 2, 2)
_BOND_DIMS = (5, 6, 2)
_D = 128
_N_NODES = 10000
_N_EDGES = 320000
_NF = len(_ATOM_DIMS)
_EF = len(_BOND_DIMS)

_NODE_OFF = tuple(int(sum(_ATOM_DIMS[:i])) for i in range(_NF))
_NCAT_ROWS = int(sum(_ATOM_DIMS))
_ECAT_ROWS = int(sum(_BOND_DIMS))
_ECOMB_ROWS = _BOND_DIMS[0] * _BOND_DIMS[1] * _BOND_DIMS[2]

_NC = 2
_NS = 16
_NW = _NC * _NS

_EPW = _N_EDGES // _NW
_ECHUNK = 80
_ENCHUNKS = _EPW // _ECHUNK

_NPW = 400
_N_NODE_WORKERS = _N_NODES // _NPW

_LANES = 16
_EOUT = _ECHUNK * _D
_EIDX = _ECHUNK * _EF


def _sc_body(nidx_hbm, eidx_hbm, ncat_hbm, ecat_hbm, vnode_hbm, vedge_hbm,
             ncat_v, ecat_v, ecomb_v, eidx_v, nidx_v, eout_v, nout_v,
             sin0, sin1, sout0, sout1):
    wid = lax.axis_index("s") * _NC + lax.axis_index("c")
    iota = lax.iota(jnp.int32, _LANES)
    ebase = wid * _EPW

    def diag(k):
        return jnp.bitwise_and(iota + k, _LANES - 1)

    pltpu.sync_copy(ncat_hbm, ncat_v)
    pltpu.sync_copy(ecat_hbm, ecat_v)

    def build_comb(t, carry):
        a = t // 12
        r = t % 12
        b = r // 2
        c = r % 2
        rowa = jnp.full((_LANES,), a * _D, jnp.int32)
        rowb = jnp.full((_LANES,), (b + _BOND_DIMS[0]) * _D, jnp.int32)
        rowc = jnp.full((_LANES,), (c + _BOND_DIMS[0] + _BOND_DIMS[1]) * _D,
                        jnp.int32)
        rowt = jnp.full((_LANES,), t * _D, jnp.int32)
        for j in range(_D // _LANES):
            col = iota + (j * _LANES)
            v = (plsc.load_gather(ecat_v, [rowa + col])
                 + plsc.load_gather(ecat_v, [rowb + col])
                 + plsc.load_gather(ecat_v, [rowc + col]))
            plsc.store_scatter(ecomb_v, [rowt + col], v)
        return carry

    lax.fori_loop(0, _ECOMB_ROWS, build_comb, 0)

    def idx_copy(ch, slot, sem):
        src = eidx_hbm.at[pl.ds((ebase + ch * _ECHUNK) * _EF, _EIDX)]
        return pltpu.make_async_copy(src, eidx_v.at[pl.ds(slot * _EIDX, _EIDX)],
                                     sem)

    def out_copy(ch, slot, sem):
        dst = vedge_hbm.at[pl.ds((ebase + ch * _ECHUNK) * _D, _EOUT)]
        return pltpu.make_async_copy(eout_v.at[pl.ds(slot * _EOUT, _EOUT)],
                                     dst, sem)

    idx_copy(0, 0, sin0).start()

    def edge_chunk(ch, carry):
        p = jnp.bitwise_and(ch, 1)
        off_idx = p * _EIDX
        off_out = p * _EOUT

        @pl.when(p == 0)
        def _():
            idx_copy(ch, 0, sin0).wait()

        @pl.when(p == 1)
        def _():
            idx_copy(ch, 1, sin1).wait()

        @pl.when(jnp.logical_and(ch + 1 < _ENCHUNKS, p == 0))
        def _():
            idx_copy(ch + 1, 1, sin1).start()

        @pl.when(jnp.logical_and(ch + 1 < _ENCHUNKS, p == 1))
        def _():
            idx_copy(ch + 1, 0, sin0).start()

        @pl.when(jnp.logical_and(ch >= 2, p == 0))
        def _():
            out_copy(ch, 0, sout0).wait()

        @pl.when(jnp.logical_and(ch >= 2, p == 1))
        def _():
            out_copy(ch, 1, sout1).wait()

        def group(g, carry2):
            rows = iota + g * _LANES
            ridx = (rows + off_idx // _EF) * _EF
            i0 = plsc.load_gather(eidx_v, [ridx])
            i1 = plsc.load_gather(eidx_v, [ridx + 1])
            i2 = plsc.load_gather(eidx_v, [ridx + 2])
            cidb = (i0 * 12 + i1 * 2 + i2) * _D
            rowb = rows * _D + off_out

            def col_block(j, carry3):
                src = cidb + j * _LANES
                dst = rowb + j * _LANES
                for k in range(_LANES):
                    d = diag(k)
                    v = plsc.load_gather(ecomb_v, [src + d])
                    plsc.store_scatter(eout_v, [dst + d], v)
                return carry3

            lax.fori_loop(0, _D // _LANES, col_block, 0)
            return carry2

        lax.fori_loop(0, _ECHUNK // _LANES, group, 0)

        @pl.when(p == 0)
        def _():
            out_copy(ch, 0, sout0).start()

        @pl.when(p == 1)
        def _():
            out_copy(ch, 1, sout1).start()

        return carry

    lax.fori_loop(0, _ENCHUNKS, edge_chunk, 0)
    out_copy(_ENCHUNKS - 2, 0, sout0).wait()
    out_copy(_ENCHUNKS - 1, 1, sout1).wait()

    @pl.when(wid < _N_NODE_WORKERS)
    def node_phase():
        base = wid * _NPW
        pltpu.sync_copy(nidx_hbm.at[pl.ds(base * _NF, _NPW * _NF)], nidx_v)

        def group(g, carry2):
            rows = iota + g * _LANES
            ivs = [
                (plsc.load_gather(nidx_v, [rows * _NF + i]) + _NODE_OFF[i]) * _D
                for i in range(_NF)
            ]
            rowb = rows * _D

            def col_block(j, carry3):
                dst = rowb + j * _LANES
                jl = j * _LANES
                for k in range(_LANES):
                    d = diag(k)
                    off = jl + d
                    acc = plsc.load_gather(ncat_v, [ivs[0] + off])
                    for i in range(1, _NF):
                        acc = acc + plsc.load_gather(ncat_v, [ivs[i] + off])
                    plsc.store_scatter(nout_v, [dst + d], acc)
                return carry3

            lax.fori_loop(0, _D // _LANES, col_block, 0)
            return carry2

        lax.fori_loop(0, _NPW // _LANES, group, 0)
        pltpu.sync_copy(nout_v, vnode_hbm.at[pl.ds(base * _D, _NPW * _D)])


@jax.jit
def _sc_call(nidx_flat, eidx_flat, ncat_flat, ecat_flat):
    mesh = plsc.VectorSubcoreMesh(core_axis_name="c", subcore_axis_name="s")
    f = pl.kernel(
        _sc_body,
        out_type=(
            jax.ShapeDtypeStruct((_N_NODES * _D,), jnp.float32),
            jax.ShapeDtypeStruct((_N_EDGES * _D,), jnp.float32),
        ),
        mesh=mesh,
        compiler_params=pltpu.CompilerParams(needs_layout_passes=False),
        scratch_types=[
            pltpu.VMEM((_NCAT_ROWS * _D,), jnp.float32),
            pltpu.VMEM((_ECAT_ROWS * _D,), jnp.float32),
            pltpu.VMEM((_ECOMB_ROWS * _D,), jnp.float32),
            pltpu.VMEM((2 * _EIDX,), jnp.int32),
            pltpu.VMEM((_NPW * _NF,), jnp.int32),
            pltpu.VMEM((2 * _EOUT,), jnp.float32),
            pltpu.VMEM((_NPW * _D,), jnp.float32),
            pltpu.SemaphoreType.DMA,
            pltpu.SemaphoreType.DMA,
            pltpu.SemaphoreType.DMA,
            pltpu.SemaphoreType.DMA,
        ],
    )
    return f(nidx_flat, eidx_flat, ncat_flat, ecat_flat)


def kernel(dNodeAttr, dEdgeAttr, node_tables, edge_tables):
    ncat = jnp.concatenate(node_tables, axis=0).reshape(-1)
    ecat = jnp.concatenate(edge_tables, axis=0).reshape(-1)
    vnode, vedge = _sc_call(dNodeAttr.reshape(-1), dEdgeAttr.reshape(-1),
                            ncat, ecat)
    return (vnode.reshape(_N_NODES, _D), vedge.reshape(_N_EDGES, _D))


# speedup vs baseline: 5.4928x; 1.4314x over previous
import jax
import jax.numpy as jnp
from jax import lax
from jax.experimental import pallas as pl
from jax.experimental.pallas import tpu as pltpu
from jax.experimental.pallas import tpu_sc as plsc

_ATOM_DIMS = (119, 5, 12, 12, 10, 6, 6, 2, 2)
_BOND_DIMS = (5, 6, 2)
_D = 128
_N_NODES = 10000
_N_EDGES = 320000
_NF = len(_ATOM_DIMS)

_NODE_OFF = tuple(int(sum(_ATOM_DIMS[:i])) for i in range(_NF))
_NCAT_ROWS = int(sum(_ATOM_DIMS))
_ECOMB_ROWS = _BOND_DIMS[0] * _BOND_DIMS[1] * _BOND_DIMS[2]

_NC = 2
_NS = 16
_NW = _NC * _NS

_EPW = _N_EDGES // _NW
_ECHUNK = 80
_ENCHUNKS = _EPW // _ECHUNK

_NPW = 400
_N_NODE_WORKERS = _N_NODES // _NPW

_LANES = 16
_EOUT = _ECHUNK * _D


def _sc_body(nidx_hbm, eidx_hbm, t_hbm, vnode_hbm, vedge_hbm,
             ncat_v, ecat_v, ecomb_v, eidx_v, nidx_v, eout_v, nout_v,
             sin0, sin1, sout0, sout1):
    wid = lax.axis_index("s") * _NC + lax.axis_index("c")
    iota = lax.iota(jnp.int32, _LANES)
    ebase = wid * _EPW

    def diag(k):
        return jnp.bitwise_and(iota + k, _LANES - 1)

    for i in range(_NF):
        pltpu.sync_copy(t_hbm[i], ncat_v.at[pl.ds(_NODE_OFF[i] * _D,
                                                  _ATOM_DIMS[i] * _D)])
    eoff = (0, _BOND_DIMS[0], _BOND_DIMS[0] + _BOND_DIMS[1])
    for i in range(3):
        pltpu.sync_copy(t_hbm[_NF + i], ecat_v.at[pl.ds(eoff[i] * _D,
                                                        _BOND_DIMS[i] * _D)])

    def build_comb(t, carry):
        a = t // 12
        r = t % 12
        b = r // 2
        c = r % 2
        rowa = jnp.full((_LANES,), a * _D, jnp.int32)
        rowb = jnp.full((_LANES,), (b + _BOND_DIMS[0]) * _D, jnp.int32)
        rowc = jnp.full((_LANES,), (c + _BOND_DIMS[0] + _BOND_DIMS[1]) * _D,
                        jnp.int32)
        rowt = jnp.full((_LANES,), t * _D, jnp.int32)
        for j in range(_D // _LANES):
            col = iota + (j * _LANES)
            v = (plsc.load_gather(ecat_v, [rowa + col])
                 + plsc.load_gather(ecat_v, [rowb + col])
                 + plsc.load_gather(ecat_v, [rowc + col]))
            plsc.store_scatter(ecomb_v, [rowt + col], v)
        return carry

    lax.fori_loop(0, _ECOMB_ROWS, build_comb, 0)

    def idx_copy(ch, slot, sem):
        src = eidx_hbm.at[pl.ds(ebase + ch * _ECHUNK, _ECHUNK)]
        return pltpu.make_async_copy(
            src, eidx_v.at[pl.ds(slot * _ECHUNK, _ECHUNK)], sem)

    def out_copy(ch, slot, sem):
        dst = vedge_hbm.at[pl.ds((ebase + ch * _ECHUNK) * _D, _EOUT)]
        return pltpu.make_async_copy(eout_v.at[pl.ds(slot * _EOUT, _EOUT)],
                                     dst, sem)

    idx_copy(0, 0, sin0).start()

    def edge_chunk(ch, carry):
        p = jnp.bitwise_and(ch, 1)
        off_idx = p * _ECHUNK
        off_out = p * _EOUT

        @pl.when(p == 0)
        def _():
            idx_copy(ch, 0, sin0).wait()

        @pl.when(p == 1)
        def _():
            idx_copy(ch, 1, sin1).wait()

        @pl.when(jnp.logical_and(ch + 1 < _ENCHUNKS, p == 0))
        def _():
            idx_copy(ch + 1, 1, sin1).start()

        @pl.when(jnp.logical_and(ch + 1 < _ENCHUNKS, p == 1))
        def _():
            idx_copy(ch + 1, 0, sin0).start()

        @pl.when(jnp.logical_and(ch >= 2, p == 0))
        def _():
            out_copy(ch, 0, sout0).wait()

        @pl.when(jnp.logical_and(ch >= 2, p == 1))
        def _():
            out_copy(ch, 1, sout1).wait()

        def group(g, carry2):
            rows = iota + g * _LANES
            cid = plsc.load_gather(eidx_v, [rows + off_idx])
            cidb = cid * _D
            rowb = rows * _D + off_out

            def col_block(j, carry3):
                src = cidb + j * _LANES
                dst = rowb + j * _LANES
                for k in range(_LANES):
                    d = diag(k)
                    v = plsc.load_gather(ecomb_v, [src + d])
                    plsc.store_scatter(eout_v, [dst + d], v)
                return carry3

            lax.fori_loop(0, _D // _LANES, col_block, 0)
            return carry2

        lax.fori_loop(0, _ECHUNK // _LANES, group, 0)

        @pl.when(p == 0)
        def _():
            out_copy(ch, 0, sout0).start()

        @pl.when(p == 1)
        def _():
            out_copy(ch, 1, sout1).start()

        return carry

    lax.fori_loop(0, _ENCHUNKS, edge_chunk, 0)
    out_copy(_ENCHUNKS - 2, 0, sout0).wait()
    out_copy(_ENCHUNKS - 1, 1, sout1).wait()

    @pl.when(wid < _N_NODE_WORKERS)
    def node_phase():
        base = wid * _NPW
        pltpu.sync_copy(nidx_hbm.at[pl.ds(base * _NF, _NPW * _NF)], nidx_v)

        def group(g, carry2):
            rows = iota + g * _LANES
            ivs = [
                (plsc.load_gather(nidx_v, [rows * _NF + i]) + _NODE_OFF[i]) * _D
                for i in range(_NF)
            ]
            rowb = rows * _D

            def col_block(j, carry3):
                dst = rowb + j * _LANES
                jl = j * _LANES
                for k in range(_LANES):
                    d = diag(k)
                    off = jl + d
                    acc = plsc.load_gather(ncat_v, [ivs[0] + off])
                    for i in range(1, _NF):
                        acc = acc + plsc.load_gather(ncat_v, [ivs[i] + off])
                    plsc.store_scatter(nout_v, [dst + d], acc)
                return carry3

            lax.fori_loop(0, _D // _LANES, col_block, 0)
            return carry2

        lax.fori_loop(0, _NPW // _LANES, group, 0)
        pltpu.sync_copy(nout_v, vnode_hbm.at[pl.ds(base * _D, _NPW * _D)])


@jax.jit
def _sc_call(nidx_flat, ecid, tables):
    mesh = plsc.VectorSubcoreMesh(core_axis_name="c", subcore_axis_name="s")
    f = pl.kernel(
        _sc_body,
        out_type=(
            jax.ShapeDtypeStruct((_N_NODES * _D,), jnp.float32),
            jax.ShapeDtypeStruct((_N_EDGES * _D,), jnp.float32),
        ),
        mesh=mesh,
        compiler_params=pltpu.CompilerParams(needs_layout_passes=False),
        scratch_types=[
            pltpu.VMEM((_NCAT_ROWS * _D,), jnp.float32),
            pltpu.VMEM((int(sum(_BOND_DIMS)) * _D,), jnp.float32),
            pltpu.VMEM((_ECOMB_ROWS * _D,), jnp.float32),
            pltpu.VMEM((2 * _ECHUNK,), jnp.int32),
            pltpu.VMEM((_NPW * _NF,), jnp.int32),
            pltpu.VMEM((2 * _EOUT,), jnp.float32),
            pltpu.VMEM((_NPW * _D,), jnp.float32),
            pltpu.SemaphoreType.DMA,
            pltpu.SemaphoreType.DMA,
            pltpu.SemaphoreType.DMA,
            pltpu.SemaphoreType.DMA,
        ],
    )
    return f(nidx_flat, ecid, tables)


def kernel(dNodeAttr, dEdgeAttr, node_tables, edge_tables):
    ecid = dEdgeAttr[:, 0] * 12 + dEdgeAttr[:, 1] * 2 + dEdgeAttr[:, 2]
    tables = tuple(t.reshape(-1) for t in node_tables + edge_tables)
    vnode, vedge = _sc_call(dNodeAttr.reshape(-1), ecid, tables)
    return (vnode.reshape(_N_NODES, _D), vedge.reshape(_N_EDGES, _D))


# speedup vs baseline: 5.5807x; 1.0160x over previous
import jax
import jax.numpy as jnp
from jax import lax
from jax.experimental import pallas as pl
from jax.experimental.pallas import tpu as pltpu
from jax.experimental.pallas import tpu_sc as plsc

_ATOM_DIMS = (119, 5, 12, 12, 10, 6, 6, 2, 2)
_BOND_DIMS = (5, 6, 2)
_D = 128
_N_NODES = 10000
_N_EDGES = 320000
_NF = len(_ATOM_DIMS)

_NODE_OFF = tuple(int(sum(_ATOM_DIMS[:i])) for i in range(_NF))
_NCAT_ROWS = int(sum(_ATOM_DIMS))
_ECOMB_ROWS = _BOND_DIMS[0] * _BOND_DIMS[1] * _BOND_DIMS[2]
_ECOMB_PAD = 64

_NC = 2
_NS = 16
_NW = _NC * _NS

_EPW = _N_EDGES // _NW
_ECHUNK = 80
_ENCHUNKS = _EPW // _ECHUNK

_NPW = 400
_N_NODE_WORKERS = _N_NODES // _NPW
_NGROUPS = _NPW // 16

_LANES = 16


def _sc_body(nidx_hbm, eidx_hbm, t_hbm, vnode_hbm, vedge_hbm, ecomb_hbm,
             ncat_v, ecat_v, ecomb_v, eidx0, eidx1, nidx_v,
             eout0, eout1, nout_v,
             sin0, sin1, sg0, sg1, sout0, sout1):
    wid = lax.axis_index("s") * _NC + lax.axis_index("c")
    iota = lax.iota(jnp.int32, _LANES)
    ebase = wid * _EPW
    is_node_worker = wid < _N_NODE_WORKERS

    def diag(k):
        return jnp.bitwise_and(iota + k, _LANES - 1)

    for i in range(_NF):
        pltpu.sync_copy(t_hbm[i], ncat_v.at[pl.ds(_NODE_OFF[i] * _D,
                                                  _ATOM_DIMS[i] * _D)])
    eoff = (0, _BOND_DIMS[0], _BOND_DIMS[0] + _BOND_DIMS[1])
    for i in range(3):
        pltpu.sync_copy(t_hbm[_NF + i], ecat_v.at[pl.ds(eoff[i] * _D,
                                                        _BOND_DIMS[i] * _D)])

    def build_comb(t, carry):
        a = t // 12
        r = t % 12
        b = r // 2
        c = r % 2
        rowa = jnp.full((_LANES,), a * _D, jnp.int32)
        rowb = jnp.full((_LANES,), (b + _BOND_DIMS[0]) * _D, jnp.int32)
        rowc = jnp.full((_LANES,), (c + _BOND_DIMS[0] + _BOND_DIMS[1]) * _D,
                        jnp.int32)
        rowt = jnp.full((_LANES,), t, jnp.int32)
        for j in range(_D // _LANES):
            col = iota + (j * _LANES)
            v = (plsc.load_gather(ecat_v, [rowa + col])
                 + plsc.load_gather(ecat_v, [rowb + col])
                 + plsc.load_gather(ecat_v, [rowc + col]))
            plsc.store_scatter(ecomb_v, [rowt, col], v)
        return carry

    lax.fori_loop(0, _ECOMB_ROWS, build_comb, 0)
    pltpu.sync_copy(ecomb_v, ecomb_hbm.at[pl.ds(wid * _ECOMB_PAD,
                                                _ECOMB_PAD)])

    nbase = wid * _NPW

    @pl.when(is_node_worker)
    def _():
        pltpu.sync_copy(nidx_hbm.at[pl.ds(nbase * _NF, _NPW * _NF)], nidx_v)

    def node_group(g):
        rows = iota + g * _LANES
        ivs = [
            (plsc.load_gather(nidx_v, [rows * _NF + i]) + _NODE_OFF[i]) * _D
            for i in range(_NF)
        ]
        rowb = rows * _D

        def col_block(j, carry3):
            dst = rowb + j * _LANES
            jl = j * _LANES
            for k in range(_LANES):
                d = diag(k)
                off = jl + d
                acc = plsc.load_gather(ncat_v, [ivs[0] + off])
                for i in range(1, _NF):
                    acc = acc + plsc.load_gather(ncat_v, [ivs[i] + off])
                plsc.store_scatter(nout_v, [dst + d], acc)
            return carry3

        lax.fori_loop(0, _D // _LANES, col_block, 0)

    row_off = jnp.full((_LANES,), wid * _ECOMB_PAD, jnp.int32)

    def idx_copy(ch, eidx_p, sem):
        src = eidx_hbm.at[pl.ds(ebase + ch * _ECHUNK, _ECHUNK)]
        return pltpu.make_async_copy(src, eidx_p, sem)

    def gather_copy(eidx_p, eout_p, sem):
        return pltpu.make_async_copy(ecomb_hbm.at[eidx_p], eout_p, sem)

    def out_copy(ch, eout_p, sem):
        dst = vedge_hbm.at[pl.ds(ebase + ch * _ECHUNK, _ECHUNK)]
        return pltpu.make_async_copy(eout_p, dst, sem)

    idx_copy(0, eidx0, sin0).start()
    idx_copy(1, eidx1, sin1).start()

    def edge_chunk(ch, carry):
        p = jnp.bitwise_and(ch, 1)

        def do_slot(eidx_p, eout_p, sin, sg, sout):
            idx_copy(ch, eidx_p, sin).wait()

            @pl.when(ch >= 2)
            def _():
                out_copy(ch, eout_p, sout).wait()

            for t in range(_ECHUNK // _LANES):
                a = iota + t * _LANES
                v = plsc.load_gather(eidx_p, [a]) + row_off
                plsc.store_scatter(eidx_p, [a], v)

            gather_copy(eidx_p, eout_p, sg).start()

            @pl.when(jnp.logical_and(is_node_worker, ch < _NGROUPS))
            def _():
                node_group(ch)

            gather_copy(eidx_p, eout_p, sg).wait()

            @pl.when(ch + 2 < _ENCHUNKS)
            def _():
                idx_copy(ch + 2, eidx_p, sin).start()

            out_copy(ch, eout_p, sout).start()

        @pl.when(p == 0)
        def _():
            do_slot(eidx0, eout0, sin0, sg0, sout0)

        @pl.when(p == 1)
        def _():
            do_slot(eidx1, eout1, sin1, sg1, sout1)

        return carry

    lax.fori_loop(0, _ENCHUNKS, edge_chunk, 0)
    out_copy(_ENCHUNKS - 2, eout0, sout0).wait()
    out_copy(_ENCHUNKS - 1, eout1, sout1).wait()

    @pl.when(is_node_worker)
    def _():
        pltpu.sync_copy(nout_v, vnode_hbm.at[pl.ds(nbase * _D, _NPW * _D)])


@jax.jit
def _sc_call(nidx_flat, ecid, tables):
    mesh = plsc.VectorSubcoreMesh(core_axis_name="c", subcore_axis_name="s")
    f = pl.kernel(
        _sc_body,
        out_type=(
            jax.ShapeDtypeStruct((_N_NODES * _D,), jnp.float32),
            jax.ShapeDtypeStruct((_N_EDGES, _D), jnp.float32),
            jax.ShapeDtypeStruct((_NW * _ECOMB_PAD, _D), jnp.float32),
        ),
        mesh=mesh,
        compiler_params=pltpu.CompilerParams(needs_layout_passes=False),
        scratch_types=[
            pltpu.VMEM((_NCAT_ROWS * _D,), jnp.float32),
            pltpu.VMEM((int(sum(_BOND_DIMS)) * _D,), jnp.float32),
            pltpu.VMEM((_ECOMB_PAD, _D), jnp.float32),
            pltpu.VMEM((_ECHUNK,), jnp.int32),
            pltpu.VMEM((_ECHUNK,), jnp.int32),
            pltpu.VMEM((_NPW * _NF,), jnp.int32),
            pltpu.VMEM((_ECHUNK, _D), jnp.float32),
            pltpu.VMEM((_ECHUNK, _D), jnp.float32),
            pltpu.VMEM((_NPW * _D,), jnp.float32),
            pltpu.SemaphoreType.DMA,
            pltpu.SemaphoreType.DMA,
            pltpu.SemaphoreType.DMA,
            pltpu.SemaphoreType.DMA,
            pltpu.SemaphoreType.DMA,
            pltpu.SemaphoreType.DMA,
        ],
    )
    return f(nidx_flat, ecid, tables)


def kernel(dNodeAttr, dEdgeAttr, node_tables, edge_tables):
    ecid = dEdgeAttr[:, 0] * 12 + dEdgeAttr[:, 1] * 2 + dEdgeAttr[:, 2]
    tables = tuple(t.reshape(-1) for t in node_tables + edge_tables)
    vnode, vedge, _ = _sc_call(dNodeAttr.reshape(-1), ecid, tables)
    return (vnode.reshape(_N_NODES, _D), vedge)


# speedup vs baseline: 5.6682x; 1.0157x over previous
import jax
import jax.numpy as jnp
from jax import lax
from jax.experimental import pallas as pl
from jax.experimental.pallas import tpu as pltpu
from jax.experimental.pallas import tpu_sc as plsc

_ATOM_DIMS = (119, 5, 12, 12, 10, 6, 6, 2, 2)
_BOND_DIMS = (5, 6, 2)
_D = 128
_N_NODES = 10000
_N_EDGES = 320000
_NF = len(_ATOM_DIMS)

_NODE_OFF = tuple(int(sum(_ATOM_DIMS[:i])) for i in range(_NF))
_NCAT_ROWS = int(sum(_ATOM_DIMS))
_ECOMB_ROWS = _BOND_DIMS[0] * _BOND_DIMS[1] * _BOND_DIMS[2]
_ECOMB_PAD = 64

_NC = 2
_NS = 16
_NW = _NC * _NS

_EPW = _N_EDGES // _NW
_ECHUNK = 80
_ENCHUNKS = _EPW // _ECHUNK

_NPW = 400
_N_NODE_WORKERS = _N_NODES // _NPW
_NGROUPS = _NPW // 16

_LANES = 16


def _sc_body(nidx_hbm, eidx_hbm, t_hbm, vnode_hbm, vedge_hbm, ecomb_hbm,
             ncat_v, ecat_v, ecomb_v, eidx0, eidx1, nidx_v,
             eout0, eout1, nout_v,
             sin0, sin1, sg0, sg1, sout0, sout1):
    wid = lax.axis_index("s") * _NC + lax.axis_index("c")
    iota = lax.iota(jnp.int32, _LANES)
    ebase = wid * _EPW
    is_node_worker = wid < _N_NODE_WORKERS

    def diag(k):
        return jnp.bitwise_and(iota + k, _LANES - 1)

    for i in range(_NF):
        pltpu.sync_copy(t_hbm[i], ncat_v.at[pl.ds(_NODE_OFF[i] * _D,
                                                  _ATOM_DIMS[i] * _D)])
    eoff = (0, _BOND_DIMS[0], _BOND_DIMS[0] + _BOND_DIMS[1])
    for i in range(3):
        pltpu.sync_copy(t_hbm[_NF + i], ecat_v.at[pl.ds(eoff[i] * _D,
                                                        _BOND_DIMS[i] * _D)])

    def build_comb(t, carry):
        a = t // 12
        r = t % 12
        b = r // 2
        c = r % 2
        rowa = jnp.full((_LANES,), a * _D, jnp.int32)
        rowb = jnp.full((_LANES,), (b + _BOND_DIMS[0]) * _D, jnp.int32)
        rowc = jnp.full((_LANES,), (c + _BOND_DIMS[0] + _BOND_DIMS[1]) * _D,
                        jnp.int32)
        rowt = jnp.full((_LANES,), t, jnp.int32)
        for j in range(_D // _LANES):
            col = iota + (j * _LANES)
            v = (plsc.load_gather(ecat_v, [rowa + col])
                 + plsc.load_gather(ecat_v, [rowb + col])
                 + plsc.load_gather(ecat_v, [rowc + col]))
            plsc.store_scatter(ecomb_v, [rowt, col], v)
        return carry

    lax.fori_loop(0, _ECOMB_ROWS, build_comb, 0)
    pltpu.sync_copy(ecomb_v, ecomb_hbm.at[pl.ds(wid * _ECOMB_PAD,
                                                _ECOMB_PAD)])

    nbase = wid * _NPW

    @pl.when(is_node_worker)
    def _():
        pltpu.sync_copy(nidx_hbm.at[pl.ds(nbase * _NF, _NPW * _NF)], nidx_v)

    def node_group(g):
        rows = iota + g * _LANES
        ivs = [
            (plsc.load_gather(nidx_v, [rows * _NF + i]) + _NODE_OFF[i]) * _D
            for i in range(_NF)
        ]
        rowb = rows * _D

        def col_block(j, carry3):
            dst = rowb + j * _LANES
            jl = j * _LANES
            for k in range(_LANES):
                d = diag(k)
                off = jl + d
                acc = plsc.load_gather(ncat_v, [ivs[0] + off])
                for i in range(1, _NF):
                    acc = acc + plsc.load_gather(ncat_v, [ivs[i] + off])
                plsc.store_scatter(nout_v, [dst + d], acc)
            return carry3

        lax.fori_loop(0, _D // _LANES, col_block, 0)

    row_off = jnp.full((_LANES,), wid * _ECOMB_PAD, jnp.int32)

    def idx_copy(ch, eidx_p, sem):
        src = eidx_hbm.at[pl.ds(ebase + ch * _ECHUNK, _ECHUNK)]
        return pltpu.make_async_copy(src, eidx_p, sem)

    def gather_copy(eidx_p, eout_p, sem):
        return pltpu.make_async_copy(ecomb_hbm.at[eidx_p], eout_p, sem)

    def out_copy(ch, eout_p, sem):
        dst = vedge_hbm.at[pl.ds(ebase + ch * _ECHUNK, _ECHUNK)]
        return pltpu.make_async_copy(eout_p, dst, sem)

    idx_copy(0, eidx0, sin0).start()
    idx_copy(1, eidx1, sin1).start()

    def edge_chunk(ch, carry):
        p = jnp.bitwise_and(ch, 1)

        def do_slot(eidx_p, eout_p, eidx_q, eout_q, sin_p, sin_q,
                    sg_p, sg_q, sout_p, sout_q):
            idx_copy(ch, eidx_p, sin_p).wait()

            @pl.when(ch >= 2)
            def _():
                out_copy(ch, eout_p, sout_p).wait()

            for t in range(_ECHUNK // _LANES):
                a = iota + t * _LANES
                v = plsc.load_gather(eidx_p, [a]) + row_off
                plsc.store_scatter(eidx_p, [a], v)

            gather_copy(eidx_p, eout_p, sg_p).start()

            @pl.when(jnp.logical_and(is_node_worker, ch < _NGROUPS))
            def _():
                node_group(ch)

            @pl.when(ch >= 1)
            def _():
                gather_copy(eidx_q, eout_q, sg_q).wait()
                out_copy(ch - 1, eout_q, sout_q).start()

                @pl.when(ch + 1 < _ENCHUNKS)
                def _():
                    idx_copy(ch + 1, eidx_q, sin_q).start()

        @pl.when(p == 0)
        def _():
            do_slot(eidx0, eout0, eidx1, eout1, sin0, sin1, sg0, sg1,
                    sout0, sout1)

        @pl.when(p == 1)
        def _():
            do_slot(eidx1, eout1, eidx0, eout0, sin1, sin0, sg1, sg0,
                    sout1, sout0)

        return carry

    lax.fori_loop(0, _ENCHUNKS, edge_chunk, 0)
    gather_copy(eidx0, eout0, sg0).wait()
    out_copy(_ENCHUNKS - 1, eout0, sout0).start()
    out_copy(_ENCHUNKS - 2, eout1, sout1).wait()
    out_copy(_ENCHUNKS - 1, eout0, sout0).wait()

    @pl.when(is_node_worker)
    def _():
        pltpu.sync_copy(nout_v, vnode_hbm.at[pl.ds(nbase * _D, _NPW * _D)])


@jax.jit
def _sc_call(nidx_flat, ecid, tables):
    mesh = plsc.VectorSubcoreMesh(core_axis_name="c", subcore_axis_name="s")
    f = pl.kernel(
        _sc_body,
        out_type=(
            jax.ShapeDtypeStruct((_N_NODES * _D,), jnp.float32),
            jax.ShapeDtypeStruct((_N_EDGES, _D), jnp.float32),
            jax.ShapeDtypeStruct((_NW * _ECOMB_PAD, _D), jnp.float32),
        ),
        mesh=mesh,
        compiler_params=pltpu.CompilerParams(needs_layout_passes=False),
        scratch_types=[
            pltpu.VMEM((_NCAT_ROWS * _D,), jnp.float32),
            pltpu.VMEM((int(sum(_BOND_DIMS)) * _D,), jnp.float32),
            pltpu.VMEM((_ECOMB_PAD, _D), jnp.float32),
            pltpu.VMEM((_ECHUNK,), jnp.int32),
            pltpu.VMEM((_ECHUNK,), jnp.int32),
            pltpu.VMEM((_NPW * _NF,), jnp.int32),
            pltpu.VMEM((_ECHUNK, _D), jnp.float32),
            pltpu.VMEM((_ECHUNK, _D), jnp.float32),
            pltpu.VMEM((_NPW * _D,), jnp.float32),
            pltpu.SemaphoreType.DMA,
            pltpu.SemaphoreType.DMA,
            pltpu.SemaphoreType.DMA,
            pltpu.SemaphoreType.DMA,
            pltpu.SemaphoreType.DMA,
            pltpu.SemaphoreType.DMA,
        ],
    )
    return f(nidx_flat, ecid, tables)


def kernel(dNodeAttr, dEdgeAttr, node_tables, edge_tables):
    ecid = dEdgeAttr[:, 0] * 12 + dEdgeAttr[:, 1] * 2 + dEdgeAttr[:, 2]
    tables = tuple(t.reshape(-1) for t in node_tables + edge_tables)
    vnode, vedge, _ = _sc_call(dNodeAttr.reshape(-1), ecid, tables)
    return (vnode.reshape(_N_NODES, _D), vedge)


# speedup vs baseline: 6.3025x; 1.1119x over previous
import jax
import jax.numpy as jnp
from jax import lax
from jax.experimental import pallas as pl
from jax.experimental.pallas import tpu as pltpu
from jax.experimental.pallas import tpu_sc as plsc

_ATOM_DIMS = (119, 5, 12, 12, 10, 6, 6, 2, 2)
_BOND_DIMS = (5, 6, 2)
_D = 128
_N_NODES = 10000
_N_EDGES = 320000
_NF = len(_ATOM_DIMS)

_NODE_OFF = tuple(int(sum(_ATOM_DIMS[:i])) for i in range(_NF))
_NCAT_ROWS = int(sum(_ATOM_DIMS))
_ECOMB_ROWS = _BOND_DIMS[0] * _BOND_DIMS[1] * _BOND_DIMS[2]
_ECOMB_PAD = 64

_NC = 2
_NS = 16
_NW = _NC * _NS

_EPW = _N_EDGES // _NW
_ECHUNK = 80
_NDMA = 75
_NCOMP = 50
_COMP_BASE = _NDMA * _ECHUNK

_NPW = 400
_N_NODE_WORKERS = _N_NODES // _NPW
_NGROUPS = _NPW // 16

_LANES = 16
_GPC = _ECHUNK // _LANES


def _sc_body(nidx_hbm, eidx_hbm, t_hbm, vnode_hbm, vedge_hbm, ecomb_hbm,
             ncat_v, ecat_v, ecomb_v, ecomb_f, eidx0, eidx1, eidx_all, nidx_v,
             eout0, eout1, cout0, cout1, nout0, nout1,
             sin0, sin1, sg0, sg1, sout0, sout1, sc0, sc1, sn0, sn1):
    wid = lax.axis_index("s") * _NC + lax.axis_index("c")
    iota = lax.iota(jnp.int32, _LANES)
    ebase = wid * _EPW
    is_node_worker = wid < _N_NODE_WORKERS

    def diag(k):
        return jnp.bitwise_and(iota + k, _LANES - 1)

    for i in range(_NF):
        pltpu.sync_copy(t_hbm[i], ncat_v.at[pl.ds(_NODE_OFF[i] * _D,
                                                  _ATOM_DIMS[i] * _D)])
    eoff = (0, _BOND_DIMS[0], _BOND_DIMS[0] + _BOND_DIMS[1])
    for i in range(3):
        pltpu.sync_copy(t_hbm[_NF + i], ecat_v.at[pl.ds(eoff[i] * _D,
                                                        _BOND_DIMS[i] * _D)])

    def build_comb(t, carry):
        a = t // 12
        r = t % 12
        b = r // 2
        c = r % 2
        rowa = jnp.full((_LANES,), a * _D, jnp.int32)
        rowb = jnp.full((_LANES,), (b + _BOND_DIMS[0]) * _D, jnp.int32)
        rowc = jnp.full((_LANES,), (c + _BOND_DIMS[0] + _BOND_DIMS[1]) * _D,
                        jnp.int32)
        rowt = jnp.full((_LANES,), t, jnp.int32)
        for j in range(_D // _LANES):
            col = iota + (j * _LANES)
            v = (plsc.load_gather(ecat_v, [rowa + col])
                 + plsc.load_gather(ecat_v, [rowb + col])
                 + plsc.load_gather(ecat_v, [rowc + col]))
            plsc.store_scatter(ecomb_v, [rowt, col], v)
            plsc.store_scatter(ecomb_f, [rowt * _D + col], v)
        return carry

    lax.fori_loop(0, _ECOMB_ROWS, build_comb, 0)
    pltpu.sync_copy(ecomb_v, ecomb_hbm.at[pl.ds(wid * _ECOMB_PAD,
                                                _ECOMB_PAD)])

    pltpu.sync_copy(eidx_hbm.at[pl.ds(ebase + _COMP_BASE,
                                      _NCOMP * _ECHUNK)], eidx_all)
    nbase = wid * _NPW

    @pl.when(is_node_worker)
    def _():
        pltpu.sync_copy(nidx_hbm.at[pl.ds(nbase * _NF, _NPW * _NF)], nidx_v)

    def edge_comp_chunk(cc, cout_p, sc_p):
        for u in range(_GPC):
            rows_l = iota + u * _LANES
            cidb = plsc.load_gather(eidx_all, [cc * _ECHUNK + rows_l]) * _D

            def col_block(j, carry3):
                src = cidb + j * _LANES
                for k in range(_LANES):
                    d = diag(k)
                    v = plsc.load_gather(ecomb_f, [src + d])
                    plsc.store_scatter(cout_p, [rows_l, j * _LANES + d], v)
                return carry3

            lax.fori_loop(0, _D // _LANES, col_block, 0)
        out = vedge_hbm.at[pl.ds(ebase + _COMP_BASE + cc * _ECHUNK, _ECHUNK)]
        pltpu.make_async_copy(cout_p, out, sc_p).start()

    def node_group(g, nout_p):
        rows = iota + g * _LANES
        ivs = [
            (plsc.load_gather(nidx_v, [rows * _NF + i]) + _NODE_OFF[i]) * _D
            for i in range(_NF)
        ]
        rowb = (iota + (g % 5) * _LANES) * _D

        def col_block(j, carry3):
            dst = rowb + j * _LANES
            jl = j * _LANES
            for k in range(_LANES):
                d = diag(k)
                off = jl + d
                acc = plsc.load_gather(ncat_v, [ivs[0] + off])
                for i in range(1, _NF):
                    acc = acc + plsc.load_gather(ncat_v, [ivs[i] + off])
                plsc.store_scatter(nout_p, [dst + d], acc)
            return carry3

        lax.fori_loop(0, _D // _LANES, col_block, 0)

    def node_out_start(g, nout_p, sn_p):
        dst = vnode_hbm.at[pl.ds((nbase + (g - 4) * _LANES) * _D,
                                 _ECHUNK * _D)]
        pltpu.make_async_copy(nout_p, dst, sn_p).start()

    row_off = jnp.full((_LANES,), wid * _ECOMB_PAD, jnp.int32)

    def idx_copy(ch, eidx_p, sem):
        src = eidx_hbm.at[pl.ds(ebase + ch * _ECHUNK, _ECHUNK)]
        return pltpu.make_async_copy(src, eidx_p, sem)

    def gather_copy(eidx_p, eout_p, sem):
        return pltpu.make_async_copy(ecomb_hbm.at[eidx_p], eout_p, sem)

    def out_copy(ch, eout_p, sem):
        dst = vedge_hbm.at[pl.ds(ebase + ch * _ECHUNK, _ECHUNK)]
        return pltpu.make_async_copy(eout_p, dst, sem)

    idx_copy(0, eidx0, sin0).start()
    idx_copy(1, eidx1, sin1).start()

    def background_work(ch):
        m = lax.rem(ch, 3)

        @pl.when(m == 0)
        def _():
            g = ch // 3

            @pl.when(is_node_worker)
            def _():
                @pl.when(jnp.bitwise_and(g // 5, 1) == 0)
                def _():
                    @pl.when(jnp.logical_and(lax.rem(g, 5) == 0, g >= 10))
                    def _():
                        pltpu.make_async_copy(
                            nout0, vnode_hbm.at[pl.ds(0, _ECHUNK * _D)],
                            sn0).wait()
                    node_group(g, nout0)

                    @pl.when(lax.rem(g, 5) == 4)
                    def _():
                        node_out_start(g, nout0, sn0)

                @pl.when(jnp.bitwise_and(g // 5, 1) == 1)
                def _():
                    @pl.when(jnp.logical_and(lax.rem(g, 5) == 0, g >= 10))
                    def _():
                        pltpu.make_async_copy(
                            nout1, vnode_hbm.at[pl.ds(0, _ECHUNK * _D)],
                            sn1).wait()
                    node_group(g, nout1)

                    @pl.when(lax.rem(g, 5) == 4)
                    def _():
                        node_out_start(g, nout1, sn1)

        @pl.when(m != 0)
        def _():
            cc = (2 * ch - 2 + jnp.where(m == 2, 1, 0)) // 3

            @pl.when(jnp.bitwise_and(cc, 1) == 0)
            def _():
                @pl.when(cc >= 2)
                def _():
                    pltpu.make_async_copy(
                        cout0, vedge_hbm.at[pl.ds(0, _ECHUNK)], sc0).wait()
                edge_comp_chunk(cc, cout0, sc0)

            @pl.when(jnp.bitwise_and(cc, 1) == 1)
            def _():
                @pl.when(cc >= 2)
                def _():
                    pltpu.make_async_copy(
                        cout1, vedge_hbm.at[pl.ds(0, _ECHUNK)], sc1).wait()
                edge_comp_chunk(cc, cout1, sc1)

    def edge_chunk(ch, carry):
        p = jnp.bitwise_and(ch, 1)

        def do_slot(eidx_p, eout_p, eidx_q, eout_q, sin_p, sin_q,
                    sg_p, sg_q, sout_p, sout_q):
            idx_copy(ch, eidx_p, sin_p).wait()

            @pl.when(ch >= 2)
            def _():
                out_copy(ch, eout_p, sout_p).wait()

            for t in range(_ECHUNK // _LANES):
                a = iota + t * _LANES
                v = plsc.load_gather(eidx_p, [a]) + row_off
                plsc.store_scatter(eidx_p, [a], v)

            gather_copy(eidx_p, eout_p, sg_p).start()

            background_work(ch)

            @pl.when(ch >= 1)
            def _():
                gather_copy(eidx_q, eout_q, sg_q).wait()
                out_copy(ch - 1, eout_q, sout_q).start()

                @pl.when(ch + 1 < _NDMA)
                def _():
                    idx_copy(ch + 1, eidx_q, sin_q).start()

        @pl.when(p == 0)
        def _():
            do_slot(eidx0, eout0, eidx1, eout1, sin0, sin1, sg0, sg1,
                    sout0, sout1)

        @pl.when(p == 1)
        def _():
            do_slot(eidx1, eout1, eidx0, eout0, sin1, sin0, sg1, sg0,
                    sout1, sout0)

        return carry

    lax.fori_loop(0, _NDMA, edge_chunk, 0)
    gather_copy(eidx0, eout0, sg0).wait()
    out_copy(_NDMA - 1, eout0, sout0).start()
    out_copy(_NDMA - 2, eout1, sout1).wait()
    out_copy(_NDMA - 1, eout0, sout0).wait()
    pltpu.make_async_copy(cout0, vedge_hbm.at[pl.ds(0, _ECHUNK)], sc0).wait()
    pltpu.make_async_copy(cout1, vedge_hbm.at[pl.ds(0, _ECHUNK)], sc1).wait()

    @pl.when(is_node_worker)
    def _():
        pltpu.make_async_copy(nout1, vnode_hbm.at[pl.ds(0, _ECHUNK * _D)],
                              sn1).wait()
        pltpu.make_async_copy(nout0, vnode_hbm.at[pl.ds(0, _ECHUNK * _D)],
                              sn0).wait()


@jax.jit
def _sc_call(nidx_flat, ecid, tables):
    mesh = plsc.VectorSubcoreMesh(core_axis_name="c", subcore_axis_name="s")
    f = pl.kernel(
        _sc_body,
        out_type=(
            jax.ShapeDtypeStruct((_N_NODES * _D,), jnp.float32),
            jax.ShapeDtypeStruct((_N_EDGES, _D), jnp.float32),
            jax.ShapeDtypeStruct((_NW * _ECOMB_PAD, _D), jnp.float32),
        ),
        mesh=mesh,
        compiler_params=pltpu.CompilerParams(needs_layout_passes=False),
        scratch_types=[
            pltpu.VMEM((_NCAT_ROWS * _D,), jnp.float32),
            pltpu.VMEM((int(sum(_BOND_DIMS)) * _D,), jnp.float32),
            pltpu.VMEM((_ECOMB_PAD, _D), jnp.float32),
            pltpu.VMEM((_ECOMB_ROWS * _D,), jnp.float32),
            pltpu.VMEM((_ECHUNK,), jnp.int32),
            pltpu.VMEM((_ECHUNK,), jnp.int32),
            pltpu.VMEM((_NCOMP * _ECHUNK,), jnp.int32),
            pltpu.VMEM((_NPW * _NF,), jnp.int32),
            pltpu.VMEM((_ECHUNK, _D), jnp.float32),
            pltpu.VMEM((_ECHUNK, _D), jnp.float32),
            pltpu.VMEM((_ECHUNK, _D), jnp.float32),
            pltpu.VMEM((_ECHUNK, _D), jnp.float32),
            pltpu.VMEM((_ECHUNK * _D,), jnp.float32),
            pltpu.VMEM((_ECHUNK * _D,), jnp.float32),
        ] + [pltpu.SemaphoreType.DMA] * 10,
    )
    return f(nidx_flat, ecid, tables)


def kernel(dNodeAttr, dEdgeAttr, node_tables, edge_tables):
    ecid = dEdgeAttr[:, 0] * 12 + dEdgeAttr[:, 1] * 2 + dEdgeAttr[:, 2]
    tables = tuple(t.reshape(-1) for t in node_tables + edge_tables)
    vnode, vedge, _ = _sc_call(dNodeAttr.reshape(-1), ecid, tables)
    return (vnode.reshape(_N_NODES, _D), vedge)


# speedup vs baseline: 6.5416x; 1.0379x over previous
import jax
import jax.numpy as jnp
from jax import lax
from jax.experimental import pallas as pl
from jax.experimental.pallas import tpu as pltpu
from jax.experimental.pallas import tpu_sc as plsc

_ATOM_DIMS = (119, 5, 12, 12, 10, 6, 6, 2, 2)
_BOND_DIMS = (5, 6, 2)
_D = 128
_N_NODES = 10000
_N_EDGES = 320000
_NF = len(_ATOM_DIMS)

_NODE_OFF = tuple(int(sum(_ATOM_DIMS[:i])) for i in range(_NF))
_NCAT_ROWS = int(sum(_ATOM_DIMS))
_ECOMB_ROWS = _BOND_DIMS[0] * _BOND_DIMS[1] * _BOND_DIMS[2]
_ECOMB_PAD = 64

_NC = 2
_NS = 16
_NW = _NC * _NS

_EPW = _N_EDGES // _NW
_ECHUNK = 80
_NDMA = 63
_NCOMP = 62
_COMP_BASE = _NDMA * _ECHUNK

_NPW = 400
_N_NODE_WORKERS = _N_NODES // _NPW
_NGROUPS = _NPW // 16

_LANES = 16
_GPC = _ECHUNK // _LANES


def _sc_body(nidx_hbm, eidx_hbm, t_hbm, vnode_hbm, vedge_hbm, ecomb_hbm,
             ncat_v, ecat_v, ecomb_v, ecomb_f, eidx0, eidx1, eidx_all, nidx_v,
             eout0, eout1, cout0, cout1, nout0, nout1,
             sin0, sin1, sg0, sg1, sout0, sout1, sc0, sc1, sn0, sn1):
    wid = lax.axis_index("s") * _NC + lax.axis_index("c")
    iota = lax.iota(jnp.int32, _LANES)
    ebase = wid * _EPW
    is_node_worker = wid < _N_NODE_WORKERS

    def diag(k):
        return jnp.bitwise_and(iota + k, _LANES - 1)

    for i in range(_NF):
        pltpu.sync_copy(t_hbm[i], ncat_v.at[pl.ds(_NODE_OFF[i] * _D,
                                                  _ATOM_DIMS[i] * _D)])
    eoff = (0, _BOND_DIMS[0], _BOND_DIMS[0] + _BOND_DIMS[1])
    for i in range(3):
        pltpu.sync_copy(t_hbm[_NF + i], ecat_v.at[pl.ds(eoff[i] * _D,
                                                        _BOND_DIMS[i] * _D)])

    def build_comb(t, carry):
        a = t // 12
        r = t % 12
        b = r // 2
        c = r % 2
        rowa = jnp.full((_LANES,), a * _D, jnp.int32)
        rowb = jnp.full((_LANES,), (b + _BOND_DIMS[0]) * _D, jnp.int32)
        rowc = jnp.full((_LANES,), (c + _BOND_DIMS[0] + _BOND_DIMS[1]) * _D,
                        jnp.int32)
        rowt = jnp.full((_LANES,), t, jnp.int32)
        for j in range(_D // _LANES):
            col = iota + (j * _LANES)
            v = (plsc.load_gather(ecat_v, [rowa + col])
                 + plsc.load_gather(ecat_v, [rowb + col])
                 + plsc.load_gather(ecat_v, [rowc + col]))
            plsc.store_scatter(ecomb_v, [rowt, col], v)
            plsc.store_scatter(ecomb_f, [rowt * _D + col], v)
        return carry

    lax.fori_loop(0, _ECOMB_ROWS, build_comb, 0)
    pltpu.sync_copy(ecomb_v, ecomb_hbm.at[pl.ds(wid * _ECOMB_PAD,
                                                _ECOMB_PAD)])

    pltpu.sync_copy(eidx_hbm.at[pl.ds(ebase + _COMP_BASE,
                                      _NCOMP * _ECHUNK)], eidx_all)
    nbase = wid * _NPW

    @pl.when(is_node_worker)
    def _():
        pltpu.sync_copy(nidx_hbm.at[pl.ds(nbase * _NF, _NPW * _NF)], nidx_v)

    def edge_comp_chunk(cc, cout_p, sc_p):
        ds = [diag(k) for k in range(_LANES)]
        for u in range(_GPC):
            rows_l = iota + u * _LANES
            cidb = plsc.load_gather(eidx_all, [cc * _ECHUNK + rows_l]) * _D

            def col_block(j, carry3):
                src = cidb + j * _LANES
                jl = j * _LANES
                for k in range(_LANES):
                    d = ds[k]
                    v = plsc.load_gather(ecomb_f, [src + d])
                    plsc.store_scatter(cout_p, [rows_l, jl + d], v)
                return carry3

            lax.fori_loop(0, _D // _LANES, col_block, 0)
        out = vedge_hbm.at[pl.ds(ebase + _COMP_BASE + cc * _ECHUNK, _ECHUNK)]
        pltpu.make_async_copy(cout_p, out, sc_p).start()

    def node_group(g, nout_p):
        rows = iota + g * _LANES
        ivs = [
            (plsc.load_gather(nidx_v, [rows * _NF + i]) + _NODE_OFF[i]) * _D
            for i in range(_NF)
        ]
        rowb = (iota + (g % 5) * _LANES) * _D

        def col_block(j, carry3):
            dst = rowb + j * _LANES
            jl = j * _LANES
            for k in range(_LANES):
                d = diag(k)
                off = jl + d
                acc = plsc.load_gather(ncat_v, [ivs[0] + off])
                for i in range(1, _NF):
                    acc = acc + plsc.load_gather(ncat_v, [ivs[i] + off])
                plsc.store_scatter(nout_p, [dst + d], acc)
            return carry3

        lax.fori_loop(0, _D // _LANES, col_block, 0)

    def node_out_start(g, nout_p, sn_p):
        dst = vnode_hbm.at[pl.ds((nbase + (g - 4) * _LANES) * _D,
                                 _ECHUNK * _D)]
        pltpu.make_async_copy(nout_p, dst, sn_p).start()

    row_off = jnp.full((_LANES,), wid * _ECOMB_PAD, jnp.int32)

    def idx_copy(ch, eidx_p, sem):
        src = eidx_hbm.at[pl.ds(ebase + ch * _ECHUNK, _ECHUNK)]
        return pltpu.make_async_copy(src, eidx_p, sem)

    def gather_copy(eidx_p, eout_p, sem):
        return pltpu.make_async_copy(ecomb_hbm.at[eidx_p], eout_p, sem)

    def out_copy(ch, eout_p, sem):
        dst = vedge_hbm.at[pl.ds(ebase + ch * _ECHUNK, _ECHUNK)]
        return pltpu.make_async_copy(eout_p, dst, sem)

    idx_copy(0, eidx0, sin0).start()
    idx_copy(1, eidx1, sin1).start()

    def background_work(ch):
        @pl.when(jnp.logical_and(jnp.bitwise_and(ch, 1) == 1,
                                 ch < 2 * _NGROUPS))
        def _():
            g = ch // 2

            @pl.when(is_node_worker)
            def _():
                @pl.when(jnp.bitwise_and(g // 5, 1) == 0)
                def _():
                    @pl.when(jnp.logical_and(lax.rem(g, 5) == 0, g >= 10))
                    def _():
                        pltpu.make_async_copy(
                            nout0, vnode_hbm.at[pl.ds(0, _ECHUNK * _D)],
                            sn0).wait()
                    node_group(g, nout0)

                    @pl.when(lax.rem(g, 5) == 4)
                    def _():
                        node_out_start(g, nout0, sn0)

                @pl.when(jnp.bitwise_and(g // 5, 1) == 1)
                def _():
                    @pl.when(jnp.logical_and(lax.rem(g, 5) == 0, g >= 10))
                    def _():
                        pltpu.make_async_copy(
                            nout1, vnode_hbm.at[pl.ds(0, _ECHUNK * _D)],
                            sn1).wait()
                    node_group(g, nout1)

                    @pl.when(lax.rem(g, 5) == 4)
                    def _():
                        node_out_start(g, nout1, sn1)

        @pl.when(ch >= 1)
        def _():
            cc = ch - 1

            @pl.when(jnp.bitwise_and(cc, 1) == 0)
            def _():
                @pl.when(cc >= 2)
                def _():
                    pltpu.make_async_copy(
                        cout0, vedge_hbm.at[pl.ds(0, _ECHUNK)], sc0).wait()
                edge_comp_chunk(cc, cout0, sc0)

            @pl.when(jnp.bitwise_and(cc, 1) == 1)
            def _():
                @pl.when(cc >= 2)
                def _():
                    pltpu.make_async_copy(
                        cout1, vedge_hbm.at[pl.ds(0, _ECHUNK)], sc1).wait()
                edge_comp_chunk(cc, cout1, sc1)

    def edge_chunk(ch, carry):
        p = jnp.bitwise_and(ch, 1)

        def do_slot(eidx_p, eout_p, eidx_q, eout_q, sin_p, sin_q,
                    sg_p, sg_q, sout_p, sout_q):
            idx_copy(ch, eidx_p, sin_p).wait()

            @pl.when(ch >= 2)
            def _():
                out_copy(ch, eout_p, sout_p).wait()

            for t in range(_ECHUNK // _LANES):
                a = iota + t * _LANES
                v = plsc.load_gather(eidx_p, [a]) + row_off
                plsc.store_scatter(eidx_p, [a], v)

            gather_copy(eidx_p, eout_p, sg_p).start()

            background_work(ch)

            @pl.when(ch >= 1)
            def _():
                gather_copy(eidx_q, eout_q, sg_q).wait()
                out_copy(ch - 1, eout_q, sout_q).start()

                @pl.when(ch + 1 < _NDMA)
                def _():
                    idx_copy(ch + 1, eidx_q, sin_q).start()

        @pl.when(p == 0)
        def _():
            do_slot(eidx0, eout0, eidx1, eout1, sin0, sin1, sg0, sg1,
                    sout0, sout1)

        @pl.when(p == 1)
        def _():
            do_slot(eidx1, eout1, eidx0, eout0, sin1, sin0, sg1, sg0,
                    sout1, sout0)

        return carry

    lax.fori_loop(0, _NDMA, edge_chunk, 0)
    gather_copy(eidx0, eout0, sg0).wait()
    out_copy(_NDMA - 1, eout0, sout0).start()
    out_copy(_NDMA - 2, eout1, sout1).wait()
    out_copy(_NDMA - 1, eout0, sout0).wait()
    pltpu.make_async_copy(cout0, vedge_hbm.at[pl.ds(0, _ECHUNK)], sc0).wait()
    pltpu.make_async_copy(cout1, vedge_hbm.at[pl.ds(0, _ECHUNK)], sc1).wait()

    @pl.when(is_node_worker)
    def _():
        pltpu.make_async_copy(nout1, vnode_hbm.at[pl.ds(0, _ECHUNK * _D)],
                              sn1).wait()
        pltpu.make_async_copy(nout0, vnode_hbm.at[pl.ds(0, _ECHUNK * _D)],
                              sn0).wait()


@jax.jit
def _sc_call(nidx_flat, ecid, tables):
    mesh = plsc.VectorSubcoreMesh(core_axis_name="c", subcore_axis_name="s")
    f = pl.kernel(
        _sc_body,
        out_type=(
            jax.ShapeDtypeStruct((_N_NODES * _D,), jnp.float32),
            jax.ShapeDtypeStruct((_N_EDGES, _D), jnp.float32),
            jax.ShapeDtypeStruct((_NW * _ECOMB_PAD, _D), jnp.float32),
        ),
        mesh=mesh,
        compiler_params=pltpu.CompilerParams(needs_layout_passes=False),
        scratch_types=[
            pltpu.VMEM((_NCAT_ROWS * _D,), jnp.float32),
            pltpu.VMEM((int(sum(_BOND_DIMS)) * _D,), jnp.float32),
            pltpu.VMEM((_ECOMB_PAD, _D), jnp.float32),
            pltpu.VMEM((_ECOMB_ROWS * _D,), jnp.float32),
            pltpu.VMEM((_ECHUNK,), jnp.int32),
            pltpu.VMEM((_ECHUNK,), jnp.int32),
            pltpu.VMEM((_NCOMP * _ECHUNK,), jnp.int32),
            pltpu.VMEM((_NPW * _NF,), jnp.int32),
            pltpu.VMEM((_ECHUNK, _D), jnp.float32),
            pltpu.VMEM((_ECHUNK, _D), jnp.float32),
            pltpu.VMEM((_ECHUNK, _D), jnp.float32),
            pltpu.VMEM((_ECHUNK, _D), jnp.float32),
            pltpu.VMEM((_ECHUNK * _D,), jnp.float32),
            pltpu.VMEM((_ECHUNK * _D,), jnp.float32),
        ] + [pltpu.SemaphoreType.DMA] * 10,
    )
    return f(nidx_flat, ecid, tables)


def kernel(dNodeAttr, dEdgeAttr, node_tables, edge_tables):
    ecid = dEdgeAttr[:, 0] * 12 + dEdgeAttr[:, 1] * 2 + dEdgeAttr[:, 2]
    tables = tuple(t.reshape(-1) for t in node_tables + edge_tables)
    vnode, vedge, _ = _sc_call(dNodeAttr.reshape(-1), ecid, tables)
    return (vnode.reshape(_N_NODES, _D), vedge)


# speedup vs baseline: 6.5600x; 1.0028x over previous
import jax
import jax.numpy as jnp
from jax import lax
from jax.experimental import pallas as pl
from jax.experimental.pallas import tpu as pltpu
from jax.experimental.pallas import tpu_sc as plsc

_ATOM_DIMS = (119, 5, 12, 12, 10, 6, 6, 2, 2)
_BOND_DIMS = (5, 6, 2)
_D = 128
_N_NODES = 10000
_N_EDGES = 320000
_NF = len(_ATOM_DIMS)

_NODE_OFF = tuple(int(sum(_ATOM_DIMS[:i])) for i in range(_NF))
_NCAT_ROWS = int(sum(_ATOM_DIMS))
_ECOMB_ROWS = _BOND_DIMS[0] * _BOND_DIMS[1] * _BOND_DIMS[2]
_ECOMB_PAD = 64

_NC = 2
_NS = 16
_NW = _NC * _NS

_EPW = _N_EDGES // _NW
_ECHUNK = 80
_NDMA = 67
_NCOMP = 58
_COMP_BASE = _NDMA * _ECHUNK

_NPW = 400
_N_NODE_WORKERS = _N_NODES // _NPW
_NGROUPS = _NPW // 16

_LANES = 16
_GPC = _ECHUNK // _LANES


def _sc_body(nidx_hbm, eidx_hbm, t_hbm, vnode_hbm, vedge_hbm, ecomb_hbm,
             ncat_v, ecat_v, ecomb_v, ecomb_f, eidx0, eidx1, eidx_all, nidx_v,
             eout0, eout1, cout0, cout1, nout0, nout1,
             sin0, sin1, sg0, sg1, sout0, sout1, sc0, sc1, sn0, sn1):
    wid = lax.axis_index("s") * _NC + lax.axis_index("c")
    iota = lax.iota(jnp.int32, _LANES)
    ebase = wid * _EPW
    is_node_worker = wid < _N_NODE_WORKERS

    def diag(k):
        return jnp.bitwise_and(iota + k, _LANES - 1)

    for i in range(_NF):
        pltpu.sync_copy(t_hbm[i], ncat_v.at[pl.ds(_NODE_OFF[i] * _D,
                                                  _ATOM_DIMS[i] * _D)])
    eoff = (0, _BOND_DIMS[0], _BOND_DIMS[0] + _BOND_DIMS[1])
    for i in range(3):
        pltpu.sync_copy(t_hbm[_NF + i], ecat_v.at[pl.ds(eoff[i] * _D,
                                                        _BOND_DIMS[i] * _D)])

    def build_comb(t, carry):
        a = t // 12
        r = t % 12
        b = r // 2
        c = r % 2
        rowa = jnp.full((_LANES,), a * _D, jnp.int32)
        rowb = jnp.full((_LANES,), (b + _BOND_DIMS[0]) * _D, jnp.int32)
        rowc = jnp.full((_LANES,), (c + _BOND_DIMS[0] + _BOND_DIMS[1]) * _D,
                        jnp.int32)
        rowt = jnp.full((_LANES,), t, jnp.int32)
        for j in range(_D // _LANES):
            col = iota + (j * _LANES)
            v = (plsc.load_gather(ecat_v, [rowa + col])
                 + plsc.load_gather(ecat_v, [rowb + col])
                 + plsc.load_gather(ecat_v, [rowc + col]))
            plsc.store_scatter(ecomb_v, [rowt, col], v)
            plsc.store_scatter(ecomb_f, [rowt * _D + col], v)
        return carry

    lax.fori_loop(0, _ECOMB_ROWS, build_comb, 0)
    pltpu.sync_copy(ecomb_v, ecomb_hbm.at[pl.ds(wid * _ECOMB_PAD,
                                                _ECOMB_PAD)])

    pltpu.sync_copy(eidx_hbm.at[pl.ds(ebase + _COMP_BASE,
                                      _NCOMP * _ECHUNK)], eidx_all)
    nbase = wid * _NPW

    @pl.when(is_node_worker)
    def _():
        pltpu.sync_copy(nidx_hbm.at[pl.ds(nbase * _NF, _NPW * _NF)], nidx_v)

    def edge_comp_chunk(cc, cout_p, sc_p):
        ds = [diag(k) for k in range(_LANES)]
        for u in range(_GPC):
            rows_l = iota + u * _LANES
            cidb = plsc.load_gather(eidx_all, [cc * _ECHUNK + rows_l]) * _D

            def col_block(j, carry3):
                src = cidb + j * _LANES
                jl = j * _LANES
                for k in range(_LANES):
                    d = ds[k]
                    v = plsc.load_gather(ecomb_f, [src + d])
                    plsc.store_scatter(cout_p, [rows_l, jl + d], v)
                return carry3

            lax.fori_loop(0, _D // _LANES, col_block, 0)
        out = vedge_hbm.at[pl.ds(ebase + _COMP_BASE + cc * _ECHUNK, _ECHUNK)]
        pltpu.make_async_copy(cout_p, out, sc_p).start()

    def node_group(g, nout_p):
        rows = iota + g * _LANES
        ivs = [
            (plsc.load_gather(nidx_v, [rows * _NF + i]) + _NODE_OFF[i]) * _D
            for i in range(_NF)
        ]
        rowb = (iota + (g % 5) * _LANES) * _D

        def col_block(j, carry3):
            dst = rowb + j * _LANES
            jl = j * _LANES
            for k in range(_LANES):
                d = diag(k)
                off = jl + d
                acc = plsc.load_gather(ncat_v, [ivs[0] + off])
                for i in range(1, _NF):
                    acc = acc + plsc.load_gather(ncat_v, [ivs[i] + off])
                plsc.store_scatter(nout_p, [dst + d], acc)
            return carry3

        lax.fori_loop(0, _D // _LANES, col_block, 0)

    def node_out_start(g, nout_p, sn_p):
        dst = vnode_hbm.at[pl.ds((nbase + (g - 4) * _LANES) * _D,
                                 _ECHUNK * _D)]
        pltpu.make_async_copy(nout_p, dst, sn_p).start()

    row_off = jnp.full((_LANES,), wid * _ECOMB_PAD, jnp.int32)

    def idx_copy(ch, eidx_p, sem):
        src = eidx_hbm.at[pl.ds(ebase + ch * _ECHUNK, _ECHUNK)]
        return pltpu.make_async_copy(src, eidx_p, sem)

    def gather_copy(eidx_p, eout_p, sem):
        return pltpu.make_async_copy(ecomb_hbm.at[eidx_p], eout_p, sem)

    def out_copy(ch, eout_p, sem):
        dst = vedge_hbm.at[pl.ds(ebase + ch * _ECHUNK, _ECHUNK)]
        return pltpu.make_async_copy(eout_p, dst, sem)

    idx_copy(0, eidx0, sin0).start()
    idx_copy(1, eidx1, sin1).start()

    def background_work(ch):
        @pl.when(jnp.logical_and(jnp.bitwise_and(ch, 1) == 1,
                                 ch < 2 * _NGROUPS))
        def _():
            g = ch // 2

            @pl.when(is_node_worker)
            def _():
                @pl.when(jnp.bitwise_and(g // 5, 1) == 0)
                def _():
                    @pl.when(jnp.logical_and(lax.rem(g, 5) == 0, g >= 10))
                    def _():
                        pltpu.make_async_copy(
                            nout0, vnode_hbm.at[pl.ds(0, _ECHUNK * _D)],
                            sn0).wait()
                    node_group(g, nout0)

                    @pl.when(lax.rem(g, 5) == 4)
                    def _():
                        node_out_start(g, nout0, sn0)

                @pl.when(jnp.bitwise_and(g // 5, 1) == 1)
                def _():
                    @pl.when(jnp.logical_and(lax.rem(g, 5) == 0, g >= 10))
                    def _():
                        pltpu.make_async_copy(
                            nout1, vnode_hbm.at[pl.ds(0, _ECHUNK * _D)],
                            sn1).wait()
                    node_group(g, nout1)

                    @pl.when(lax.rem(g, 5) == 4)
                    def _():
                        node_out_start(g, nout1, sn1)

        @pl.when(jnp.logical_and(ch >= 1, ch <= _NCOMP))
        def _():
            cc = ch - 1

            @pl.when(jnp.bitwise_and(cc, 1) == 0)
            def _():
                @pl.when(cc >= 2)
                def _():
                    pltpu.make_async_copy(
                        cout0, vedge_hbm.at[pl.ds(0, _ECHUNK)], sc0).wait()
                edge_comp_chunk(cc, cout0, sc0)

            @pl.when(jnp.bitwise_and(cc, 1) == 1)
            def _():
                @pl.when(cc >= 2)
                def _():
                    pltpu.make_async_copy(
                        cout1, vedge_hbm.at[pl.ds(0, _ECHUNK)], sc1).wait()
                edge_comp_chunk(cc, cout1, sc1)

    def edge_chunk(ch, carry):
        p = jnp.bitwise_and(ch, 1)

        def do_slot(eidx_p, eout_p, eidx_q, eout_q, sin_p, sin_q,
                    sg_p, sg_q, sout_p, sout_q):
            idx_copy(ch, eidx_p, sin_p).wait()

            @pl.when(ch >= 2)
            def _():
                out_copy(ch, eout_p, sout_p).wait()

            for t in range(_ECHUNK // _LANES):
                a = iota + t * _LANES
                v = plsc.load_gather(eidx_p, [a]) + row_off
                plsc.store_scatter(eidx_p, [a], v)

            gather_copy(eidx_p, eout_p, sg_p).start()

            background_work(ch)

            @pl.when(ch >= 1)
            def _():
                gather_copy(eidx_q, eout_q, sg_q).wait()
                out_copy(ch - 1, eout_q, sout_q).start()

                @pl.when(ch + 1 < _NDMA)
                def _():
                    idx_copy(ch + 1, eidx_q, sin_q).start()

        @pl.when(p == 0)
        def _():
            do_slot(eidx0, eout0, eidx1, eout1, sin0, sin1, sg0, sg1,
                    sout0, sout1)

        @pl.when(p == 1)
        def _():
            do_slot(eidx1, eout1, eidx0, eout0, sin1, sin0, sg1, sg0,
                    sout1, sout0)

        return carry

    lax.fori_loop(0, _NDMA, edge_chunk, 0)
    gather_copy(eidx0, eout0, sg0).wait()
    out_copy(_NDMA - 1, eout0, sout0).start()
    out_copy(_NDMA - 2, eout1, sout1).wait()
    out_copy(_NDMA - 1, eout0, sout0).wait()
    pltpu.make_async_copy(cout0, vedge_hbm.at[pl.ds(0, _ECHUNK)], sc0).wait()
    pltpu.make_async_copy(cout1, vedge_hbm.at[pl.ds(0, _ECHUNK)], sc1).wait()

    @pl.when(is_node_worker)
    def _():
        pltpu.make_async_copy(nout1, vnode_hbm.at[pl.ds(0, _ECHUNK * _D)],
                              sn1).wait()
        pltpu.make_async_copy(nout0, vnode_hbm.at[pl.ds(0, _ECHUNK * _D)],
                              sn0).wait()


@jax.jit
def _sc_call(nidx_flat, ecid, tables):
    mesh = plsc.VectorSubcoreMesh(core_axis_name="c", subcore_axis_name="s")
    f = pl.kernel(
        _sc_body,
        out_type=(
            jax.ShapeDtypeStruct((_N_NODES * _D,), jnp.float32),
            jax.ShapeDtypeStruct((_N_EDGES, _D), jnp.float32),
            jax.ShapeDtypeStruct((_NW * _ECOMB_PAD, _D), jnp.float32),
        ),
        mesh=mesh,
        compiler_params=pltpu.CompilerParams(needs_layout_passes=False),
        scratch_types=[
            pltpu.VMEM((_NCAT_ROWS * _D,), jnp.float32),
            pltpu.VMEM((int(sum(_BOND_DIMS)) * _D,), jnp.float32),
            pltpu.VMEM((_ECOMB_PAD, _D), jnp.float32),
            pltpu.VMEM((_ECOMB_ROWS * _D,), jnp.float32),
            pltpu.VMEM((_ECHUNK,), jnp.int32),
            pltpu.VMEM((_ECHUNK,), jnp.int32),
            pltpu.VMEM((_NCOMP * _ECHUNK,), jnp.int32),
            pltpu.VMEM((_NPW * _NF,), jnp.int32),
            pltpu.VMEM((_ECHUNK, _D), jnp.float32),
            pltpu.VMEM((_ECHUNK, _D), jnp.float32),
            pltpu.VMEM((_ECHUNK, _D), jnp.float32),
            pltpu.VMEM((_ECHUNK, _D), jnp.float32),
            pltpu.VMEM((_ECHUNK * _D,), jnp.float32),
            pltpu.VMEM((_ECHUNK * _D,), jnp.float32),
        ] + [pltpu.SemaphoreType.DMA] * 10,
    )
    return f(nidx_flat, ecid, tables)


def kernel(dNodeAttr, dEdgeAttr, node_tables, edge_tables):
    ecid = dEdgeAttr[:, 0] * 12 + dEdgeAttr[:, 1] * 2 + dEdgeAttr[:, 2]
    tables = tuple(t.reshape(-1) for t in node_tables + edge_tables)
    vnode, vedge, _ = _sc_call(dNodeAttr.reshape(-1), ecid, tables)
    return (vnode.reshape(_N_NODES, _D), vedge)


# speedup vs baseline: 6.6424x; 1.0126x over previous
import jax
import jax.numpy as jnp
from jax import lax
from jax.experimental import pallas as pl
from jax.experimental.pallas import tpu as pltpu
from jax.experimental.pallas import tpu_sc as plsc

_ATOM_DIMS = (119, 5, 12, 12, 10, 6, 6, 2, 2)
_BOND_DIMS = (5, 6, 2)
_D = 128
_N_NODES = 10000
_N_EDGES = 320000
_NF = len(_ATOM_DIMS)

_NODE_OFF = tuple(int(sum(_ATOM_DIMS[:i])) for i in range(_NF))
_NCAT_ROWS = int(sum(_ATOM_DIMS))
_ECOMB_ROWS = _BOND_DIMS[0] * _BOND_DIMS[1] * _BOND_DIMS[2]
_ECOMB_PAD = 64

_NC = 2
_NS = 16
_NW = _NC * _NS

_EPW = _N_EDGES // _NW
_ECHUNK = 80
_NDMA = 67
_NCOMP = 58
_COMP_BASE = _NDMA * _ECHUNK

_NPW = 400
_N_NODE_WORKERS = _N_NODES // _NPW
_NGROUPS = _NPW // 16

_LANES = 16
_GPC = _ECHUNK // _LANES


def _sc_body(nidx_hbm, eidx_hbm, t_hbm, vnode_hbm, vedge_hbm, ecomb_hbm,
             ncat_v, ecat_v, ecomb_v, ecomb_f, eidx0, eidx1, eidx_all, nidx_v,
             eout0, eout1, cout0, cout1, nout0, nout1,
             sin0, sin1, sg0, sg1, sout0, sout1, sc0, sc1, sn0, sn1):
    wid = lax.axis_index("s") * _NC + lax.axis_index("c")
    iota = lax.iota(jnp.int32, _LANES)
    ebase = wid * _EPW
    is_node_worker = wid < _N_NODE_WORKERS

    def diag(k):
        return jnp.bitwise_and(iota + k, _LANES - 1)

    for i in range(_NF):
        pltpu.sync_copy(t_hbm[i], ncat_v.at[pl.ds(_NODE_OFF[i] * _D,
                                                  _ATOM_DIMS[i] * _D)])
    eoff = (0, _BOND_DIMS[0], _BOND_DIMS[0] + _BOND_DIMS[1])
    for i in range(3):
        pltpu.sync_copy(t_hbm[_NF + i], ecat_v.at[pl.ds(eoff[i] * _D,
                                                        _BOND_DIMS[i] * _D)])

    def build_comb(t, carry):
        a = t // 12
        r = t % 12
        b = r // 2
        c = r % 2
        rowa = jnp.full((_LANES,), a * _D, jnp.int32)
        rowb = jnp.full((_LANES,), (b + _BOND_DIMS[0]) * _D, jnp.int32)
        rowc = jnp.full((_LANES,), (c + _BOND_DIMS[0] + _BOND_DIMS[1]) * _D,
                        jnp.int32)
        rowt = jnp.full((_LANES,), t, jnp.int32)
        for j in range(_D // _LANES):
            col = iota + (j * _LANES)
            v = (plsc.load_gather(ecat_v, [rowa + col])
                 + plsc.load_gather(ecat_v, [rowb + col])
                 + plsc.load_gather(ecat_v, [rowc + col]))
            plsc.store_scatter(ecomb_v, [rowt, col], v)
            plsc.store_scatter(ecomb_f, [rowt * _D + col], v)
        return carry

    lax.fori_loop(0, _ECOMB_ROWS, build_comb, 0)
    pltpu.sync_copy(ecomb_v, ecomb_hbm.at[pl.ds(wid * _ECOMB_PAD,
                                                _ECOMB_PAD)])

    pltpu.sync_copy(eidx_hbm.at[pl.ds(ebase + _COMP_BASE,
                                      _NCOMP * _ECHUNK)], eidx_all)
    nbase = wid * _NPW

    @pl.when(is_node_worker)
    def _():
        pltpu.sync_copy(nidx_hbm.at[pl.ds(nbase * _NF, _NPW * _NF)], nidx_v)

    def edge_comp_chunk(cc, cout_p, sc_p):
        ds = [diag(k) for k in range(_LANES)]
        for u in range(_GPC):
            rows_l = iota + u * _LANES
            cidb = plsc.load_gather(eidx_all, [cc * _ECHUNK + rows_l]) * _D

            def col_block(j, carry3):
                src = cidb + j * _LANES
                jl = j * _LANES
                for k in range(_LANES):
                    d = ds[k]
                    v = plsc.load_gather(ecomb_f, [src + d])
                    plsc.store_scatter(cout_p, [rows_l, jl + d], v)
                return carry3

            lax.fori_loop(0, _D // _LANES, col_block, 0)
        out = vedge_hbm.at[pl.ds(ebase + _COMP_BASE + cc * _ECHUNK, _ECHUNK)]
        pltpu.make_async_copy(cout_p, out, sc_p).start()

    def node_group(g, nout_p):
        rows = iota + g * _LANES
        ivs = [
            (plsc.load_gather(nidx_v, [rows * _NF + i]) + _NODE_OFF[i]) * _D
            for i in range(_NF)
        ]
        rowb = (iota + (g % 5) * _LANES) * _D

        def col_block(j, carry3):
            dst = rowb + j * _LANES
            jl = j * _LANES
            for k in range(_LANES):
                d = diag(k)
                off = jl + d
                acc = plsc.load_gather(ncat_v, [ivs[0] + off])
                for i in range(1, _NF):
                    acc = acc + plsc.load_gather(ncat_v, [ivs[i] + off])
                plsc.store_scatter(nout_p, [dst + d], acc)
            return carry3

        lax.fori_loop(0, _D // _LANES, col_block, 0)

    def node_out_start(g, nout_p, sn_p):
        dst = vnode_hbm.at[pl.ds((nbase + (g - 4) * _LANES) * _D,
                                 _ECHUNK * _D)]
        pltpu.make_async_copy(nout_p, dst, sn_p).start()

    row_off = jnp.full((_LANES,), wid * _ECOMB_PAD, jnp.int32)

    def idx_copy(ch, eidx_p, sem):
        src = eidx_hbm.at[pl.ds(ebase + ch * _ECHUNK, _ECHUNK)]
        return pltpu.make_async_copy(src, eidx_p, sem)

    my_ecomb = ecomb_hbm.at[pl.ds(wid * _ECOMB_PAD, _ECOMB_PAD)]

    def gather_copy(eidx_p, eout_p, sem):
        return pltpu.make_async_copy(my_ecomb.at[eidx_p], eout_p, sem)

    def out_copy(ch, eout_p, sem):
        dst = vedge_hbm.at[pl.ds(ebase + ch * _ECHUNK, _ECHUNK)]
        return pltpu.make_async_copy(eout_p, dst, sem)

    idx_copy(0, eidx0, sin0).start()
    idx_copy(1, eidx1, sin1).start()

    def background_work(ch):
        @pl.when(jnp.logical_and(jnp.bitwise_and(ch, 1) == 1,
                                 ch < 2 * _NGROUPS))
        def _():
            g = ch // 2

            @pl.when(is_node_worker)
            def _():
                @pl.when(jnp.bitwise_and(g // 5, 1) == 0)
                def _():
                    @pl.when(jnp.logical_and(lax.rem(g, 5) == 0, g >= 10))
                    def _():
                        pltpu.make_async_copy(
                            nout0, vnode_hbm.at[pl.ds(0, _ECHUNK * _D)],
                            sn0).wait()
                    node_group(g, nout0)

                    @pl.when(lax.rem(g, 5) == 4)
                    def _():
                        node_out_start(g, nout0, sn0)

                @pl.when(jnp.bitwise_and(g // 5, 1) == 1)
                def _():
                    @pl.when(jnp.logical_and(lax.rem(g, 5) == 0, g >= 10))
                    def _():
                        pltpu.make_async_copy(
                            nout1, vnode_hbm.at[pl.ds(0, _ECHUNK * _D)],
                            sn1).wait()
                    node_group(g, nout1)

                    @pl.when(lax.rem(g, 5) == 4)
                    def _():
                        node_out_start(g, nout1, sn1)

        @pl.when(jnp.logical_and(ch >= 1, ch <= _NCOMP))
        def _():
            cc = ch - 1

            @pl.when(jnp.bitwise_and(cc, 1) == 0)
            def _():
                @pl.when(cc >= 2)
                def _():
                    pltpu.make_async_copy(
                        cout0, vedge_hbm.at[pl.ds(0, _ECHUNK)], sc0).wait()
                edge_comp_chunk(cc, cout0, sc0)

            @pl.when(jnp.bitwise_and(cc, 1) == 1)
            def _():
                @pl.when(cc >= 2)
                def _():
                    pltpu.make_async_copy(
                        cout1, vedge_hbm.at[pl.ds(0, _ECHUNK)], sc1).wait()
                edge_comp_chunk(cc, cout1, sc1)

    def edge_chunk(ch, carry):
        p = jnp.bitwise_and(ch, 1)

        def do_slot(eidx_p, eout_p, eidx_q, eout_q, sin_p, sin_q,
                    sg_p, sg_q, sout_p, sout_q):
            idx_copy(ch, eidx_p, sin_p).wait()

            @pl.when(ch >= 2)
            def _():
                out_copy(ch, eout_p, sout_p).wait()

            gather_copy(eidx_p, eout_p, sg_p).start()

            background_work(ch)

            @pl.when(ch >= 1)
            def _():
                gather_copy(eidx_q, eout_q, sg_q).wait()
                out_copy(ch - 1, eout_q, sout_q).start()

                @pl.when(ch + 1 < _NDMA)
                def _():
                    idx_copy(ch + 1, eidx_q, sin_q).start()

        @pl.when(p == 0)
        def _():
            do_slot(eidx0, eout0, eidx1, eout1, sin0, sin1, sg0, sg1,
                    sout0, sout1)

        @pl.when(p == 1)
        def _():
            do_slot(eidx1, eout1, eidx0, eout0, sin1, sin0, sg1, sg0,
                    sout1, sout0)

        return carry

    lax.fori_loop(0, _NDMA, edge_chunk, 0)
    gather_copy(eidx0, eout0, sg0).wait()
    out_copy(_NDMA - 1, eout0, sout0).start()
    out_copy(_NDMA - 2, eout1, sout1).wait()
    out_copy(_NDMA - 1, eout0, sout0).wait()
    pltpu.make_async_copy(cout0, vedge_hbm.at[pl.ds(0, _ECHUNK)], sc0).wait()
    pltpu.make_async_copy(cout1, vedge_hbm.at[pl.ds(0, _ECHUNK)], sc1).wait()

    @pl.when(is_node_worker)
    def _():
        pltpu.make_async_copy(nout1, vnode_hbm.at[pl.ds(0, _ECHUNK * _D)],
                              sn1).wait()
        pltpu.make_async_copy(nout0, vnode_hbm.at[pl.ds(0, _ECHUNK * _D)],
                              sn0).wait()


@jax.jit
def _sc_call(nidx_flat, ecid, tables):
    mesh = plsc.VectorSubcoreMesh(core_axis_name="c", subcore_axis_name="s")
    f = pl.kernel(
        _sc_body,
        out_type=(
            jax.ShapeDtypeStruct((_N_NODES * _D,), jnp.float32),
            jax.ShapeDtypeStruct((_N_EDGES, _D), jnp.float32),
            jax.ShapeDtypeStruct((_NW * _ECOMB_PAD, _D), jnp.float32),
        ),
        mesh=mesh,
        compiler_params=pltpu.CompilerParams(needs_layout_passes=False),
        scratch_types=[
            pltpu.VMEM((_NCAT_ROWS * _D,), jnp.float32),
            pltpu.VMEM((int(sum(_BOND_DIMS)) * _D,), jnp.float32),
            pltpu.VMEM((_ECOMB_PAD, _D), jnp.float32),
            pltpu.VMEM((_ECOMB_ROWS * _D,), jnp.float32),
            pltpu.VMEM((_ECHUNK,), jnp.int32),
            pltpu.VMEM((_ECHUNK,), jnp.int32),
            pltpu.VMEM((_NCOMP * _ECHUNK,), jnp.int32),
            pltpu.VMEM((_NPW * _NF,), jnp.int32),
            pltpu.VMEM((_ECHUNK, _D), jnp.float32),
            pltpu.VMEM((_ECHUNK, _D), jnp.float32),
            pltpu.VMEM((_ECHUNK, _D), jnp.float32),
            pltpu.VMEM((_ECHUNK, _D), jnp.float32),
            pltpu.VMEM((_ECHUNK * _D,), jnp.float32),
            pltpu.VMEM((_ECHUNK * _D,), jnp.float32),
        ] + [pltpu.SemaphoreType.DMA] * 10,
    )
    return f(nidx_flat, ecid, tables)


def kernel(dNodeAttr, dEdgeAttr, node_tables, edge_tables):
    ecid = dEdgeAttr[:, 0] * 12 + dEdgeAttr[:, 1] * 2 + dEdgeAttr[:, 2]
    tables = tuple(t.reshape(-1) for t in node_tables + edge_tables)
    vnode, vedge, _ = _sc_call(dNodeAttr.reshape(-1), ecid, tables)
    return (vnode.reshape(_N_NODES, _D), vedge)


# speedup vs baseline: 6.6818x; 1.0059x over previous
import jax
import jax.numpy as jnp
from jax import lax
from jax.experimental import pallas as pl
from jax.experimental.pallas import tpu as pltpu
from jax.experimental.pallas import tpu_sc as plsc

_ATOM_DIMS = (119, 5, 12, 12, 10, 6, 6, 2, 2)
_BOND_DIMS = (5, 6, 2)
_D = 128
_N_NODES = 10000
_N_EDGES = 320000
_NF = len(_ATOM_DIMS)

_NODE_OFF = tuple(int(sum(_ATOM_DIMS[:i])) for i in range(_NF))
_NCAT_ROWS = int(sum(_ATOM_DIMS))
_ECOMB_ROWS = _BOND_DIMS[0] * _BOND_DIMS[1] * _BOND_DIMS[2]
_ECOMB_PAD = 64

_NC = 2
_NS = 16
_NW = _NC * _NS

_EPW = _N_EDGES // _NW
_ECHUNK = 80
_NDMA = 65
_NCOMP = 60
_COMP_BASE = _NDMA * _ECHUNK

_NPW = 400
_N_NODE_WORKERS = _N_NODES // _NPW
_NGROUPS = _NPW // 16

_LANES = 16
_GPC = _ECHUNK // _LANES


def _sc_body(nidx_hbm, eidx_hbm, t_hbm, vnode_hbm, vedge_hbm, ecomb_hbm,
             ncat_v, ecat_v, ecomb_v, ecomb_f, eidx0, eidx1, eidx_all, nidx_v,
             eout0, eout1, cout0, cout1, nout0, nout1,
             sin0, sin1, sg0, sg1, sout0, sout1, sc0, sc1, sn0, sn1):
    wid = lax.axis_index("s") * _NC + lax.axis_index("c")
    iota = lax.iota(jnp.int32, _LANES)
    ebase = wid * _EPW
    is_node_worker = wid < _N_NODE_WORKERS

    def diag(k):
        return jnp.bitwise_and(iota + k, _LANES - 1)

    for i in range(_NF):
        pltpu.sync_copy(t_hbm[i], ncat_v.at[pl.ds(_NODE_OFF[i] * _D,
                                                  _ATOM_DIMS[i] * _D)])
    eoff = (0, _BOND_DIMS[0], _BOND_DIMS[0] + _BOND_DIMS[1])
    for i in range(3):
        pltpu.sync_copy(t_hbm[_NF + i], ecat_v.at[pl.ds(eoff[i] * _D,
                                                        _BOND_DIMS[i] * _D)])

    def build_comb(t, carry):
        a = t // 12
        r = t % 12
        b = r // 2
        c = r % 2
        rowa = jnp.full((_LANES,), a * _D, jnp.int32)
        rowb = jnp.full((_LANES,), (b + _BOND_DIMS[0]) * _D, jnp.int32)
        rowc = jnp.full((_LANES,), (c + _BOND_DIMS[0] + _BOND_DIMS[1]) * _D,
                        jnp.int32)
        rowt = jnp.full((_LANES,), t, jnp.int32)
        for j in range(_D // _LANES):
            col = iota + (j * _LANES)
            v = (plsc.load_gather(ecat_v, [rowa + col])
                 + plsc.load_gather(ecat_v, [rowb + col])
                 + plsc.load_gather(ecat_v, [rowc + col]))
            plsc.store_scatter(ecomb_v, [rowt, col], v)
            plsc.store_scatter(ecomb_f, [rowt * _D + col], v)
        return carry

    lax.fori_loop(0, _ECOMB_ROWS, build_comb, 0)
    pltpu.sync_copy(ecomb_v, ecomb_hbm.at[pl.ds(wid * _ECOMB_PAD,
                                                _ECOMB_PAD)])

    pltpu.sync_copy(eidx_hbm.at[pl.ds(ebase + _COMP_BASE,
                                      _NCOMP * _ECHUNK)], eidx_all)
    nbase = wid * _NPW

    @pl.when(is_node_worker)
    def _():
        pltpu.sync_copy(nidx_hbm.at[pl.ds(nbase * _NF, _NPW * _NF)], nidx_v)

    def edge_comp_chunk(cc, cout_p, sc_p):
        ds = [diag(k) for k in range(_LANES)]
        for u in range(_GPC):
            rows_l = iota + u * _LANES
            cidb = plsc.load_gather(eidx_all, [cc * _ECHUNK + rows_l]) * _D

            def col_block(j, carry3):
                src = cidb + j * _LANES
                jl = j * _LANES
                for k in range(_LANES):
                    d = ds[k]
                    v = plsc.load_gather(ecomb_f, [src + d])
                    plsc.store_scatter(cout_p, [rows_l, jl + d], v)
                return carry3

            lax.fori_loop(0, _D // _LANES, col_block, 0)
        out = vedge_hbm.at[pl.ds(ebase + _COMP_BASE + cc * _ECHUNK, _ECHUNK)]
        pltpu.make_async_copy(cout_p, out, sc_p).start()

    def node_group(g, nout_p):
        rows = iota + g * _LANES
        ivs = [
            (plsc.load_gather(nidx_v, [rows * _NF + i]) + _NODE_OFF[i]) * _D
            for i in range(_NF)
        ]
        rowb = (iota + (g % 5) * _LANES) * _D

        def col_block(j, carry3):
            dst = rowb + j * _LANES
            jl = j * _LANES
            for k in range(_LANES):
                d = diag(k)
                off = jl + d
                acc = plsc.load_gather(ncat_v, [ivs[0] + off])
                for i in range(1, _NF):
                    acc = acc + plsc.load_gather(ncat_v, [ivs[i] + off])
                plsc.store_scatter(nout_p, [dst + d], acc)
            return carry3

        lax.fori_loop(0, _D // _LANES, col_block, 0)

    def node_out_start(g, nout_p, sn_p):
        dst = vnode_hbm.at[pl.ds((nbase + (g - 4) * _LANES) * _D,
                                 _ECHUNK * _D)]
        pltpu.make_async_copy(nout_p, dst, sn_p).start()

    row_off = jnp.full((_LANES,), wid * _ECOMB_PAD, jnp.int32)

    def idx_copy(ch, eidx_p, sem):
        src = eidx_hbm.at[pl.ds(ebase + ch * _ECHUNK, _ECHUNK)]
        return pltpu.make_async_copy(src, eidx_p, sem)

    my_ecomb = ecomb_hbm.at[pl.ds(wid * _ECOMB_PAD, _ECOMB_PAD)]

    def gather_copy(eidx_p, eout_p, sem):
        return pltpu.make_async_copy(my_ecomb.at[eidx_p], eout_p, sem)

    def out_copy(ch, eout_p, sem):
        dst = vedge_hbm.at[pl.ds(ebase + ch * _ECHUNK, _ECHUNK)]
        return pltpu.make_async_copy(eout_p, dst, sem)

    idx_copy(0, eidx0, sin0).start()
    idx_copy(1, eidx1, sin1).start()

    def background_work(ch):
        @pl.when(jnp.logical_and(jnp.bitwise_and(ch, 1) == 1,
                                 ch < 2 * _NGROUPS))
        def _():
            g = ch // 2

            @pl.when(is_node_worker)
            def _():
                @pl.when(jnp.bitwise_and(g // 5, 1) == 0)
                def _():
                    @pl.when(jnp.logical_and(lax.rem(g, 5) == 0, g >= 10))
                    def _():
                        pltpu.make_async_copy(
                            nout0, vnode_hbm.at[pl.ds(0, _ECHUNK * _D)],
                            sn0).wait()
                    node_group(g, nout0)

                    @pl.when(lax.rem(g, 5) == 4)
                    def _():
                        node_out_start(g, nout0, sn0)

                @pl.when(jnp.bitwise_and(g // 5, 1) == 1)
                def _():
                    @pl.when(jnp.logical_and(lax.rem(g, 5) == 0, g >= 10))
                    def _():
                        pltpu.make_async_copy(
                            nout1, vnode_hbm.at[pl.ds(0, _ECHUNK * _D)],
                            sn1).wait()
                    node_group(g, nout1)

                    @pl.when(lax.rem(g, 5) == 4)
                    def _():
                        node_out_start(g, nout1, sn1)

        @pl.when(jnp.logical_and(ch >= 1, ch <= _NCOMP))
        def _():
            cc = ch - 1

            @pl.when(jnp.bitwise_and(cc, 1) == 0)
            def _():
                @pl.when(cc >= 2)
                def _():
                    pltpu.make_async_copy(
                        cout0, vedge_hbm.at[pl.ds(0, _ECHUNK)], sc0).wait()
                edge_comp_chunk(cc, cout0, sc0)

            @pl.when(jnp.bitwise_and(cc, 1) == 1)
            def _():
                @pl.when(cc >= 2)
                def _():
                    pltpu.make_async_copy(
                        cout1, vedge_hbm.at[pl.ds(0, _ECHUNK)], sc1).wait()
                edge_comp_chunk(cc, cout1, sc1)

    def edge_chunk(ch, carry):
        p = jnp.bitwise_and(ch, 1)

        def do_slot(eidx_p, eout_p, eidx_q, eout_q, sin_p, sin_q,
                    sg_p, sg_q, sout_p, sout_q):
            idx_copy(ch, eidx_p, sin_p).wait()

            @pl.when(ch >= 2)
            def _():
                out_copy(ch, eout_p, sout_p).wait()

            gather_copy(eidx_p, eout_p, sg_p).start()

            background_work(ch)

            @pl.when(ch >= 1)
            def _():
                gather_copy(eidx_q, eout_q, sg_q).wait()
                out_copy(ch - 1, eout_q, sout_q).start()

                @pl.when(ch + 1 < _NDMA)
                def _():
                    idx_copy(ch + 1, eidx_q, sin_q).start()

        @pl.when(p == 0)
        def _():
            do_slot(eidx0, eout0, eidx1, eout1, sin0, sin1, sg0, sg1,
                    sout0, sout1)

        @pl.when(p == 1)
        def _():
            do_slot(eidx1, eout1, eidx0, eout0, sin1, sin0, sg1, sg0,
                    sout1, sout0)

        return carry

    lax.fori_loop(0, _NDMA, edge_chunk, 0)
    gather_copy(eidx0, eout0, sg0).wait()
    out_copy(_NDMA - 1, eout0, sout0).start()
    out_copy(_NDMA - 2, eout1, sout1).wait()
    out_copy(_NDMA - 1, eout0, sout0).wait()
    pltpu.make_async_copy(cout0, vedge_hbm.at[pl.ds(0, _ECHUNK)], sc0).wait()
    pltpu.make_async_copy(cout1, vedge_hbm.at[pl.ds(0, _ECHUNK)], sc1).wait()

    @pl.when(is_node_worker)
    def _():
        pltpu.make_async_copy(nout1, vnode_hbm.at[pl.ds(0, _ECHUNK * _D)],
                              sn1).wait()
        pltpu.make_async_copy(nout0, vnode_hbm.at[pl.ds(0, _ECHUNK * _D)],
                              sn0).wait()


@jax.jit
def _sc_call(nidx_flat, ecid, tables):
    mesh = plsc.VectorSubcoreMesh(core_axis_name="c", subcore_axis_name="s")
    f = pl.kernel(
        _sc_body,
        out_type=(
            jax.ShapeDtypeStruct((_N_NODES * _D,), jnp.float32),
            jax.ShapeDtypeStruct((_N_EDGES, _D), jnp.float32),
            jax.ShapeDtypeStruct((_NW * _ECOMB_PAD, _D), jnp.float32),
        ),
        mesh=mesh,
        compiler_params=pltpu.CompilerParams(needs_layout_passes=False),
        scratch_types=[
            pltpu.VMEM((_NCAT_ROWS * _D,), jnp.float32),
            pltpu.VMEM((int(sum(_BOND_DIMS)) * _D,), jnp.float32),
            pltpu.VMEM((_ECOMB_PAD, _D), jnp.float32),
            pltpu.VMEM((_ECOMB_ROWS * _D,), jnp.float32),
            pltpu.VMEM((_ECHUNK,), jnp.int32),
            pltpu.VMEM((_ECHUNK,), jnp.int32),
            pltpu.VMEM((_NCOMP * _ECHUNK,), jnp.int32),
            pltpu.VMEM((_NPW * _NF,), jnp.int32),
            pltpu.VMEM((_ECHUNK, _D), jnp.float32),
            pltpu.VMEM((_ECHUNK, _D), jnp.float32),
            pltpu.VMEM((_ECHUNK, _D), jnp.float32),
            pltpu.VMEM((_ECHUNK, _D), jnp.float32),
            pltpu.VMEM((_ECHUNK * _D,), jnp.float32),
            pltpu.VMEM((_ECHUNK * _D,), jnp.float32),
        ] + [pltpu.SemaphoreType.DMA] * 10,
    )
    return f(nidx_flat, ecid, tables)


def kernel(dNodeAttr, dEdgeAttr, node_tables, edge_tables):
    ecid = dEdgeAttr[:, 0] * 12 + dEdgeAttr[:, 1] * 2 + dEdgeAttr[:, 2]
    tables = tuple(t.reshape(-1) for t in node_tables + edge_tables)
    vnode, vedge, _ = _sc_call(dNodeAttr.reshape(-1), ecid, tables)
    return (vnode.reshape(_N_NODES, _D), vedge)
